# Initial kernel scaffold; baseline (speedup 1.0000x reference)
import jax
import jax.numpy as jnp
from jax.experimental import pallas as pl


def kernel(x, W1, a_src1, a_dst1, b1, gamma, beta, W2, a_src2, a_dst2, b2, embedding_w, logit_p, edge_index_ppi, edge_index_homolog, y, train_mask):
    raise NotImplementedError("write your pallas kernel here")


# speedup vs baseline: 1.6584x; 1.6584x over previous
import functools
import jax
import jax.numpy as jnp
import numpy as np
from jax import lax
from jax.experimental import pallas as pl
from jax.experimental.pallas import tpu as pltpu

EPS = float(np.finfo(float).eps)
N = 10000
D = 128
HID = 64
OUTC = 2

ROW_BLK = 1024
COL_BLK = 1024


def _leaky(x, slope):
    return jnp.where(x >= 0, x, slope * x)


def _gram_body(colscale_ref, xb_ref, xc_ref, out_ref):
    colscale = colscale_ref[...]
    xb = xb_ref[...] * colscale
    xc = xc_ref[...] * colscale
    ones = jnp.ones((1, D), jnp.float32)
    rowsumc = jax.lax.dot_general(ones, xc, (((1,), (1,)), ((), ())),
                                  preferred_element_type=jnp.float32)
    gram = jax.lax.dot_general(xb, xc, (((1,), (1,)), ((), ())),
                               preferred_element_type=jnp.float32)
    out_ref[...] = gram / (rowsumc + 1e-6)


def _gram_pallas(x, colscale):
    gr = (N + ROW_BLK - 1) // ROW_BLK
    gc = (N + COL_BLK - 1) // COL_BLK
    return pl.pallas_call(
        _gram_body,
        grid=(gr, gc),
        in_specs=[
            pl.BlockSpec((1, D), lambda i, j: (0, 0)),
            pl.BlockSpec((ROW_BLK, D), lambda i, j: (i, 0)),
            pl.BlockSpec((COL_BLK, D), lambda i, j: (j, 0)),
        ],
        out_specs=pl.BlockSpec((ROW_BLK, COL_BLK), lambda i, j: (i, j)),
        out_shape=jax.ShapeDtypeStruct((N, N), jnp.float32),
    )(colscale, x, x)


def _gat_layer(x, src, dst, W, a_src, a_dst, b):
    h = x @ W
    e = (h @ a_src)[src] + (h @ a_dst)[dst]
    e = _leaky(e, 0.2)
    w = jnp.exp(e)
    denom = jax.ops.segment_sum(w, dst, num_segments=N)
    num = jax.ops.segment_sum(h[src] * w[:, None], dst, num_segments=N)
    return num / (denom[:, None] + 1e-16) + b


def _graph_cnn(x, src, dst, p, y, train_mask):
    (W1, a_src1, a_dst1, b1, gamma, beta, W2, a_src2, a_dst2, b2) = p
    hid = _gat_layer(x, src, dst, W1, a_src1, a_dst1, b1)
    mean = hid.mean(axis=0)
    var = hid.var(axis=0)
    hid = (hid - mean) / jnp.sqrt(var + 1e-5) * gamma + beta
    hid = _leaky(hid, 0.01)
    out = _gat_layer(hid, src, dst, W2, a_src2, a_dst2, b2)
    out = jax.nn.log_softmax(out, axis=1)
    picked = out[jnp.arange(out.shape[0]), y]
    maskf = train_mask.astype(out.dtype)
    loss = -(picked * maskf).sum() / maskf.sum()
    return out, loss


def kernel(x, W1, a_src1, a_dst1, b1, gamma, beta, W2, a_src2, a_dst2, b2,
           embedding_w, logit_p, edge_index_ppi, edge_index_homolog, y,
           train_mask):
    out = jnp.zeros((N,), x.dtype)
    loss = jnp.zeros((1,), x.dtype)
    edges = [edge_index_ppi, edge_index_homolog]
    num = W1.shape[0]
    for i in range(num):
        p = (W1[i], a_src1[i], a_dst1[i], b1[i], gamma[i], beta[i],
             W2[i], a_src2[i], a_dst2[i], b2[i])
        ei = edges[i % 2]
        temp, tl = _graph_cnn(x, ei[0], ei[1], p, y, train_mask)
        out = out + jnp.exp(temp[:, 1])
        loss = loss + tl
    node_p = out / num
    loss = loss / num

    dropout_p = jax.nn.sigmoid(logit_p)
    unif = 0.5
    approx = (jnp.log(dropout_p + EPS) - jnp.log(1.0 - dropout_p + EPS)
              + jnp.log(unif + EPS) - jnp.log(1.0 - unif + EPS))
    approx_output = jax.nn.sigmoid(approx / 0.1)
    pw_vimp = 1.0 - dropout_p
    ew = jax.nn.sigmoid(embedding_w)
    colscale = ((1.0 - approx_output) * ew).reshape(1, D)

    graph = _gram_pallas(x, colscale)

    A = x * colscale
    rowsum = A.sum(axis=1)
    v = (2.0 * node_p - 1.0) / (rowsum + 1e-6)
    t = A.T @ v
    diff = A @ t
    norm1 = jax.nn.sigmoid(diff)

    return (node_p, loss, norm1, graph, pw_vimp)


# speedup vs baseline: 37.3628x; 22.5288x over previous
import functools
import jax
import jax.numpy as jnp
import numpy as np
from jax import lax
from jax.experimental import pallas as pl
from jax.experimental.pallas import tpu as pltpu
from jax.experimental.pallas import tpu_sc as plsc

EPS = float(np.finfo(float).eps)
N = 10000
D = 128
HID = 64
E = 320000

NC = 2
NS = 16
NW = NC * NS
EPT = E // NW
CHUNK = 80
NCH = EPT // CHUNK
NPT = N // NS
NP_PAD = 10240
DPT = NP_PAD // NS

ROW_BLK = 1024
COL_BLK = 1024

_f32 = jnp.float32
_i32 = jnp.int32


def _leaky(x, slope):
    return jnp.where(x >= 0, x, slope * x)


def _sig(z):
    return 1.0 / (1.0 + jnp.exp(-z))


def _sc_l1_body(h0_hbm, h1_hbm, es_hbm, ed_hbm, src_hbm, dst_hbm,
                num_out, den_out,
                es_v, ed_v, src_v, dst_v, w_v, rows_v, znum_v, zden_v,
                num_sh, den_sh, sem):
    c = lax.axis_index("c")
    s = lax.axis_index("s")
    wid = c * NS + s

    def zrow(r, carry):
        for q in range(4):
            znum_v[r, pl.ds(q * 16, 16)] = jnp.zeros((16,), _f32)
        return carry
    lax.fori_loop(0, 128, zrow, 0)

    def zd(i, carry):
        zden_v[pl.ds(i * 16, 16)] = jnp.zeros((16,), _f32)
        return carry
    lax.fori_loop(0, 40, zd, 0)

    rsd = pl.ds(s * DPT, DPT)
    for m, h_hbm in enumerate((h0_hbm, h1_hbm)):
        for j in range(5):
            pltpu.sync_copy(znum_v, num_sh.at[pl.ds(s * DPT + j * 128, 128)])
        pltpu.sync_copy(zden_v, den_sh.at[rsd])
        plsc.subcore_barrier()

        pltpu.sync_copy(es_hbm.at[m], es_v)
        pltpu.sync_copy(ed_hbm.at[m], ed_v)
        pltpu.sync_copy(src_hbm.at[m].at[wid], src_v)
        pltpu.sync_copy(dst_hbm.at[m].at[wid], dst_v)

        def chunk_body(ci, carry):
            cp = pltpu.async_copy(h_hbm.at[src_v.at[ci]], rows_v, sem)
            for i in range(CHUNK // 16):
                sl = pl.ds(i * 16, 16)
                s16 = src_v[ci, sl]
                d16 = dst_v[ci, sl]
                a = (plsc.load_gather(es_v, [s16])
                     + plsc.load_gather(ed_v, [d16]))
                a = jnp.where(a >= 0, a, 0.2 * a)
                w_v[sl] = jnp.exp(a)
            cp.wait()

            def scale_body(e, carry2):
                ws = plsc.load_gather(w_v, [jnp.zeros((16,), _i32) + e])
                for q in range(HID // 16):
                    sl = pl.ds(q * 16, 16)
                    rows_v[e, sl] = rows_v[e, sl] * ws
                return carry2
            lax.fori_loop(0, CHUNK, scale_body, 0)

            pltpu.sync_copy(rows_v, num_sh.at[dst_v.at[ci]], add=True)
            pltpu.sync_copy(w_v, den_sh.at[dst_v.at[ci]], add=True)
            return carry
        lax.fori_loop(0, NCH, chunk_body, 0)

        plsc.subcore_barrier()
        pltpu.sync_copy(num_sh.at[rsd], num_out.at[c].at[m].at[rsd])
        pltpu.sync_copy(den_sh.at[rsd], den_out.at[c].at[m].at[rsd])


def _sc_l1(h0, h1, es, ed, src, dst):
    mesh = plsc.VectorSubcoreMesh(core_axis_name="c", subcore_axis_name="s")
    f = functools.partial(
        pl.kernel, _sc_l1_body, mesh=mesh,
        compiler_params=pltpu.CompilerParams(needs_layout_passes=False, use_tc_tiling_on_sc=False),
        out_type=[jax.ShapeDtypeStruct((NC, 2, NP_PAD, HID), _f32),
                  jax.ShapeDtypeStruct((NC, 2, NP_PAD), _f32)],
        scratch_types=[
            pltpu.VMEM((N,), _f32),
            pltpu.VMEM((N,), _f32),
            pltpu.VMEM((NCH, CHUNK), _i32),
            pltpu.VMEM((NCH, CHUNK), _i32),
            pltpu.VMEM((CHUNK,), _f32),
            pltpu.VMEM((CHUNK, HID), _f32),
            pltpu.VMEM((128, HID), _f32),
            pltpu.VMEM((640,), _f32),
            pltpu.VMEM_SHARED((NP_PAD, HID), _f32),
            pltpu.VMEM_SHARED((NP_PAD,), _f32),
            pltpu.SemaphoreType.DMA,
        ])()
    return f(h0, h1, es, ed, src, dst)


def _sc_l2_body(h2_hbm, prm_hbm, src_hbm, dst_hbm,
                n0_out, n1_out, den_out,
                es_v, ed_v, h2_v, src_v, dst_v, w_v, m0_v, m1_v, zden_v,
                prm_v,
                n0a_sh, n0b_sh, n1a_sh, n1b_sh, dena_sh, denb_sh):
    c = lax.axis_index("c")
    s = lax.axis_index("s")
    wid = c * NS + s

    def zd(i, carry):
        zden_v[pl.ds(i * 16, 16)] = jnp.zeros((16,), _f32)
        return carry
    lax.fori_loop(0, 40, zd, 0)
    for sh in (n0a_sh, n0b_sh, n1a_sh, n1b_sh, dena_sh, denb_sh):
        pltpu.sync_copy(zden_v, sh.at[pl.ds(s * DPT, DPT)])
    plsc.subcore_barrier()

    z16 = jnp.zeros((16,), _i32)
    iot = lax.iota(_i32, 16)
    for m, (n0_sh, n1_sh, den_sh) in enumerate(
            ((n0a_sh, n1a_sh, dena_sh), (n0b_sh, n1b_sh, denb_sh))):
        pltpu.sync_copy(h2_hbm.at[m], h2_v)
        pltpu.sync_copy(prm_hbm.at[m], prm_v)
        pltpu.sync_copy(src_hbm.at[m].at[wid], src_v)
        pltpu.sync_copy(dst_hbm.at[m].at[wid], dst_v)
        as0 = plsc.load_gather(prm_v, [z16 + 1])
        as1 = plsc.load_gather(prm_v, [z16 + 2])
        ad0 = plsc.load_gather(prm_v, [z16 + 3])
        ad1 = plsc.load_gather(prm_v, [z16 + 4])

        def node_body(i, carry):
            idx16 = iot + i * 16
            c0 = plsc.load_gather(h2_v, [idx16, z16])
            c1 = plsc.load_gather(h2_v, [idx16, z16 + 1])
            sl = pl.ds(i * 16, 16)
            es_v[sl] = as0 * c0 + as1 * c1
            ed_v[sl] = ad0 * c0 + ad1 * c1
            return carry
        lax.fori_loop(0, N // 16, node_body, 0)

        def chunk_body(ci, carry):
            for i in range(CHUNK // 16):
                sl = pl.ds(i * 16, 16)
                s16 = src_v[ci, sl]
                d16 = dst_v[ci, sl]
                a = (plsc.load_gather(es_v, [s16])
                     + plsc.load_gather(ed_v, [d16]))
                a = jnp.where(a >= 0, a, 0.2 * a)
                w16 = jnp.exp(a)
                w_v[sl] = w16
                m0_v[sl] = w16 * plsc.load_gather(h2_v, [s16, z16])
                m1_v[sl] = w16 * plsc.load_gather(h2_v, [s16, z16 + 1])
            pltpu.sync_copy(w_v, den_sh.at[dst_v.at[ci]], add=True)
            pltpu.sync_copy(m0_v, n0_sh.at[dst_v.at[ci]], add=True)
            pltpu.sync_copy(m1_v, n1_sh.at[dst_v.at[ci]], add=True)
            return carry
        lax.fori_loop(0, NCH, chunk_body, 0)

    plsc.subcore_barrier()
    rs = pl.ds(s * DPT, DPT)
    pltpu.sync_copy(n0a_sh.at[rs], n0_out.at[c].at[0].at[rs])
    pltpu.sync_copy(n0b_sh.at[rs], n0_out.at[c].at[1].at[rs])
    pltpu.sync_copy(n1a_sh.at[rs], n1_out.at[c].at[0].at[rs])
    pltpu.sync_copy(n1b_sh.at[rs], n1_out.at[c].at[1].at[rs])
    pltpu.sync_copy(dena_sh.at[rs], den_out.at[c].at[0].at[rs])
    pltpu.sync_copy(denb_sh.at[rs], den_out.at[c].at[1].at[rs])


def _sc_l2(h2, prm, src, dst):
    mesh = plsc.VectorSubcoreMesh(core_axis_name="c", subcore_axis_name="s")
    f = functools.partial(
        pl.kernel, _sc_l2_body, mesh=mesh,
        compiler_params=pltpu.CompilerParams(needs_layout_passes=False, use_tc_tiling_on_sc=False),
        out_type=[jax.ShapeDtypeStruct((NC, 2, NP_PAD), _f32),
                  jax.ShapeDtypeStruct((NC, 2, NP_PAD), _f32),
                  jax.ShapeDtypeStruct((NC, 2, NP_PAD), _f32)],
        scratch_types=[
            pltpu.VMEM((N,), _f32),
            pltpu.VMEM((N,), _f32),
            pltpu.VMEM((N, 2), _f32),
            pltpu.VMEM((NCH, CHUNK), _i32),
            pltpu.VMEM((NCH, CHUNK), _i32),
            pltpu.VMEM((CHUNK,), _f32),
            pltpu.VMEM((CHUNK,), _f32),
            pltpu.VMEM((CHUNK,), _f32),
            pltpu.VMEM((640,), _f32),
            pltpu.VMEM((8,), _f32),
            pltpu.VMEM_SHARED((NP_PAD,), _f32),
            pltpu.VMEM_SHARED((NP_PAD,), _f32),
            pltpu.VMEM_SHARED((NP_PAD,), _f32),
            pltpu.VMEM_SHARED((NP_PAD,), _f32),
            pltpu.VMEM_SHARED((NP_PAD,), _f32),
            pltpu.VMEM_SHARED((NP_PAD,), _f32),
        ])()
    return f(h2, prm, src, dst)


def _tca_body(x_ref, W1_ref, as_ref, ad_ref,
              h0_ref, h1_ref, es0_ref, es1_ref, ed0_ref, ed1_ref):
    x = x_ref[...]
    outs = ((h0_ref, es0_ref, ed0_ref), (h1_ref, es1_ref, ed1_ref))
    for m in range(2):
        h = jnp.dot(x, W1_ref[m], preferred_element_type=_f32)
        h_ref, es_ref, ed_ref = outs[m]
        h_ref[...] = h
        es_ref[...] = jnp.dot(h, as_ref[m], preferred_element_type=_f32)
        ed_ref[...] = jnp.dot(h, ad_ref[m], preferred_element_type=_f32)


def _tca(x, W1, a_src1, a_dst1):
    blk = 1000
    grid = N // blk
    outs = [jax.ShapeDtypeStruct((N, HID), _f32)] * 2 + \
           [jax.ShapeDtypeStruct((N, 1), _f32)] * 4
    return pl.pallas_call(
        _tca_body,
        grid=(grid,),
        in_specs=[
            pl.BlockSpec((blk, D), lambda i: (i, 0)),
            pl.BlockSpec((2, D, HID), lambda i: (0, 0, 0)),
            pl.BlockSpec((2, HID, 1), lambda i: (0, 0, 0)),
            pl.BlockSpec((2, HID, 1), lambda i: (0, 0, 0)),
        ],
        out_specs=[pl.BlockSpec((blk, HID), lambda i: (i, 0))] * 2
                  + [pl.BlockSpec((blk, 1), lambda i: (i, 0))] * 4,
        out_shape=outs,
    )(x, W1, a_src1.reshape(2, HID, 1), a_dst1.reshape(2, HID, 1))


def _tcb_body(num_ref, den_ref, b1_ref, g_ref, be_ref, W2_ref, h2_ref):
    num = num_ref[0, 0] + num_ref[1, 0]
    den = den_ref[0, 0, 0] + den_ref[1, 0, 0]
    dcol = den.reshape(N, 1)
    hid = num / (dcol + 1e-16) + b1_ref[0]
    mean = jnp.mean(hid, axis=0, keepdims=True)
    var = jnp.mean((hid - mean) ** 2, axis=0, keepdims=True)
    hid = (hid - mean) / jnp.sqrt(var + 1e-5) * g_ref[0] + be_ref[0]
    hid = _leaky(hid, 0.01)
    h2_ref[0] = jnp.dot(hid, W2_ref[0], preferred_element_type=_f32)


def _tcb(num1, den1, b1, gamma, beta, W2):
    return pl.pallas_call(
        _tcb_body,
        grid=(2,),
        in_specs=[
            pl.BlockSpec((NC, 1, N, HID), lambda m: (0, m, 0, 0)),
            pl.BlockSpec((NC, 1, 1, N), lambda m: (0, m, 0, 0)),
            pl.BlockSpec((1, 1, HID), lambda m: (m, 0, 0)),
            pl.BlockSpec((1, 1, HID), lambda m: (m, 0, 0)),
            pl.BlockSpec((1, 1, HID), lambda m: (m, 0, 0)),
            pl.BlockSpec((1, HID, 2), lambda m: (m, 0, 0)),
        ],
        out_specs=pl.BlockSpec((1, N, 2), lambda m: (m, 0, 0)),
        out_shape=jax.ShapeDtypeStruct((2, N, 2), _f32),
    )(num1, den1.reshape(NC, 2, 1, N), b1.reshape(2, 1, HID),
      gamma.reshape(2, 1, HID), beta.reshape(2, 1, HID), W2)


def _tcc_body(n0_ref, n1_ref, den_ref, b2_ref, y_ref, mask_ref,
              np_ref, loss_ref):
    y = y_ref[...]
    maskf = mask_ref[...]
    msum = jnp.sum(maskf)
    np_acc = jnp.zeros((N,), _f32)
    loss_acc = jnp.zeros((), _f32)
    for m in range(2):
        den = den_ref[0, m] + den_ref[1, m]
        o0 = (n0_ref[0, m] + n0_ref[1, m]) / (den + 1e-16) + b2_ref[m, 0]
        o1 = (n1_ref[0, m] + n1_ref[1, m]) / (den + 1e-16) + b2_ref[m, 1]
        mx = jnp.maximum(o0, o1)
        l0 = o0 - mx
        l1 = o1 - mx
        lse = jnp.log(jnp.exp(l0) + jnp.exp(l1))
        lsm0 = l0 - lse
        lsm1 = l1 - lse
        np_acc = np_acc + jnp.exp(lsm1)
        picked = jnp.where(y == 1, lsm1, lsm0)
        loss_acc = loss_acc - jnp.sum(picked * maskf) / msum
    np_ref[...] = np_acc * 0.5
    loss_ref[...] = jnp.broadcast_to(loss_acc * 0.5, (1, 1))


def _tcc(n0, n1, den2, b2, y, maskf):
    full = lambda *shape: pl.BlockSpec(shape, lambda: tuple(0 for _ in shape))
    return pl.pallas_call(
        _tcc_body,
        in_specs=[full(NC, 2, N), full(NC, 2, N), full(NC, 2, N),
                  full(2, 2), full(N), full(N)],
        out_specs=[full(N), full(1, 1)],
        out_shape=[jax.ShapeDtypeStruct((N,), _f32),
                   jax.ShapeDtypeStruct((1, 1), _f32)],
    )(n0, n1, den2, b2, y, maskf)


def _tcd_body(x_ref, lp_ref, ew_ref, np_ref,
              norm_ref, cs_ref, pw_ref):
    lp = lp_ref[...]
    dp = _sig(lp)
    approx = (jnp.log(dp + EPS) - jnp.log(1.0 - dp + EPS)
              + jnp.log(0.5 + EPS) - jnp.log(1.0 - 0.5 + EPS))
    ao = _sig(approx / 0.1)
    ew = _sig(ew_ref[...])
    cs = (1.0 - ao) * ew
    A = x_ref[...] * cs
    rowsum = jnp.sum(A, axis=1, keepdims=True)
    v = (2.0 * np_ref[...] - 1.0) / (rowsum + 1e-6)
    t = jnp.sum(A * v, axis=0, keepdims=True)
    diff = jnp.sum(A * t, axis=1, keepdims=True)
    norm_ref[...] = _sig(diff)
    cs_ref[...] = cs
    pw_ref[...] = 1.0 - dp


def _tcd(x, logit_p, embedding_w, np_col):
    full = lambda *shape: pl.BlockSpec(shape, lambda: tuple(0 for _ in shape))
    return pl.pallas_call(
        _tcd_body,
        in_specs=[full(N, D), full(1, D), full(1, D), full(N, 1)],
        out_specs=[full(N, 1), full(1, D), full(1, D)],
        out_shape=[jax.ShapeDtypeStruct((N, 1), _f32),
                   jax.ShapeDtypeStruct((1, D), _f32),
                   jax.ShapeDtypeStruct((1, D), _f32)],
    )(x, logit_p.reshape(1, D), embedding_w.reshape(1, D), np_col)


def _gram_body(colscale_ref, xb_ref, xc_ref, out_ref):
    colscale = colscale_ref[...]
    xb = xb_ref[...] * colscale
    xc = xc_ref[...] * colscale
    ones = jnp.ones((1, D), _f32)
    rowsumc = jax.lax.dot_general(ones, xc, (((1,), (1,)), ((), ())),
                                  preferred_element_type=_f32)
    gram = jax.lax.dot_general(xb, xc, (((1,), (1,)), ((), ())),
                               preferred_element_type=_f32)
    out_ref[...] = gram / (rowsumc + 1e-6)


def _gram_pallas(x, colscale):
    gr = (N + ROW_BLK - 1) // ROW_BLK
    gc = (N + COL_BLK - 1) // COL_BLK
    return pl.pallas_call(
        _gram_body,
        grid=(gr, gc),
        in_specs=[
            pl.BlockSpec((1, D), lambda i, j: (0, 0)),
            pl.BlockSpec((ROW_BLK, D), lambda i, j: (i, 0)),
            pl.BlockSpec((COL_BLK, D), lambda i, j: (j, 0)),
        ],
        out_specs=pl.BlockSpec((ROW_BLK, COL_BLK), lambda i, j: (i, j)),
        out_shape=jax.ShapeDtypeStruct((N, N), _f32),
    )(colscale, x, x)


def kernel(x, W1, a_src1, a_dst1, b1, gamma, beta, W2, a_src2, a_dst2, b2,
           embedding_w, logit_p, edge_index_ppi, edge_index_homolog, y,
           train_mask):
    src = jnp.stack([edge_index_ppi[0].reshape(NW, NCH, CHUNK),
                     edge_index_homolog[0].reshape(NW, NCH, CHUNK)])
    dst = jnp.stack([edge_index_ppi[1].reshape(NW, NCH, CHUNK),
                     edge_index_homolog[1].reshape(NW, NCH, CHUNK)])

    h0, h1, es0, es1, ed0, ed1 = _tca(x, W1, a_src1, a_dst1)
    es1c = jnp.stack([es0.reshape(N), es1.reshape(N)])
    ed1c = jnp.stack([ed0.reshape(N), ed1.reshape(N)])

    num1p, den1p = _sc_l1(h0, h1, es1c, ed1c, src, dst)
    num1 = num1p[:, :, :N]
    den1 = den1p[:, :, :N]

    h2 = _tcb(num1, den1, b1, gamma, beta, W2)
    prm = jnp.concatenate(
        [jnp.zeros((2, 1), _f32), a_src2, a_dst2,
         jnp.zeros((2, 3), _f32)], axis=1)

    n0p, n1p, den2p = _sc_l2(h2, prm, src, dst)
    n0, n1, den2 = n0p[:, :, :N], n1p[:, :, :N], den2p[:, :, :N]

    node_p, loss2d = _tcc(n0, n1, den2, b2, y,
                          train_mask.astype(_f32))
    loss = loss2d.reshape(1)

    norm_col, colscale, pw2d = _tcd(x, logit_p, embedding_w,
                                    node_p.reshape(N, 1))
    norm1 = norm_col.reshape(N)
    pw_vimp = pw2d.reshape(D)

    graph = _gram_pallas(x, colscale)

    return (node_p, loss, norm1, graph, pw_vimp)


# speedup vs baseline: 37.7508x; 1.0104x over previous
import functools
import jax
import jax.numpy as jnp
import numpy as np
from jax import lax
from jax.experimental import pallas as pl
from jax.experimental.pallas import tpu as pltpu
from jax.experimental.pallas import tpu_sc as plsc

EPS = float(np.finfo(float).eps)
N = 10000
D = 128
HID = 64
E = 320000

NC = 2
NS = 16
NW = NC * NS
EPT = E // NW
CHUNK = 80
NCH = EPT // CHUNK
NPT = N // NS
NP_PAD = 10240
DPT = NP_PAD // NS

ROW_BLK = 1024
COL_BLK = 1024

_f32 = jnp.float32
_i32 = jnp.int32


def _leaky(x, slope):
    return jnp.where(x >= 0, x, slope * x)


def _sig(z):
    return 1.0 / (1.0 + jnp.exp(-z))


def _sc_l1_body(h0_hbm, h1_hbm, es_hbm, ed_hbm, src_hbm, dst_hbm,
                num_out, den_out,
                es_v, ed_v, src_v, dst_v, w_v, rows_v, rows_b, znum_v,
                zden_v, num_sh, den_sh, sem, sem_b):
    c = lax.axis_index("c")
    s = lax.axis_index("s")
    wid = c * NS + s

    def zrow(r, carry):
        for q in range(4):
            znum_v[r, pl.ds(q * 16, 16)] = jnp.zeros((16,), _f32)
        return carry
    lax.fori_loop(0, 128, zrow, 0)

    def zd(i, carry):
        zden_v[pl.ds(i * 16, 16)] = jnp.zeros((16,), _f32)
        return carry
    lax.fori_loop(0, 40, zd, 0)

    rsd = pl.ds(s * DPT, DPT)
    for m, h_hbm in enumerate((h0_hbm, h1_hbm)):
        for j in range(5):
            pltpu.sync_copy(znum_v, num_sh.at[pl.ds(s * DPT + j * 128, 128)])
        pltpu.sync_copy(zden_v, den_sh.at[rsd])
        plsc.subcore_barrier()

        pltpu.sync_copy(es_hbm.at[m], es_v)
        pltpu.sync_copy(ed_hbm.at[m], ed_v)
        pltpu.sync_copy(src_hbm.at[m].at[wid], src_v)
        pltpu.sync_copy(dst_hbm.at[m].at[wid], dst_v)

        def scalar_phase(ci):
            for i in range(CHUNK // 16):
                sl = pl.ds(i * 16, 16)
                s16 = src_v[ci, sl]
                d16 = dst_v[ci, sl]
                a = (plsc.load_gather(es_v, [s16])
                     + plsc.load_gather(ed_v, [d16]))
                a = jnp.where(a >= 0, a, 0.2 * a)
                w_v[sl] = jnp.exp(a)

        def scale_rows(rows):
            def scale_g(g, carry2):
                base = g * 16
                w16 = w_v[pl.ds(base, 16)]
                for j in range(16):
                    ws = lax.gather(
                        w16, jnp.full((16, 1), j, _i32),
                        lax.GatherDimensionNumbers(
                            offset_dims=(), collapsed_slice_dims=(0,),
                            start_index_map=(0,)),
                        (1,), mode=lax.GatherScatterMode.PROMISE_IN_BOUNDS)
                    e = base + j
                    for q in range(HID // 16):
                        sl = pl.ds(q * 16, 16)
                        rows[e, sl] = rows[e, sl] * ws
                return carry2
            lax.fori_loop(0, CHUNK // 16, scale_g, 0)

        def scatter(rows, ci):
            pltpu.sync_copy(rows, num_sh.at[dst_v.at[ci]], add=True)
            pltpu.sync_copy(w_v, den_sh.at[dst_v.at[ci]], add=True)

        cp0 = pltpu.async_copy(h_hbm.at[src_v.at[0]], rows_v, sem)

        def pair_body(k, carry):
            ca = 2 * k
            cb = 2 * k + 1
            scalar_phase(ca)
            cpb = pltpu.async_copy(h_hbm.at[src_v.at[cb]], rows_b, sem_b)
            pltpu.make_async_copy(h_hbm.at[src_v.at[0]], rows_v, sem).wait()
            scale_rows(rows_v)
            scatter(rows_v, ca)
            scalar_phase(cb)
            cpn = pltpu.async_copy(h_hbm.at[src_v.at[cb + 1]], rows_v, sem)
            cpb.wait()
            scale_rows(rows_b)
            scatter(rows_b, cb)
            return carry
        lax.fori_loop(0, (NCH - 1) // 2, pair_body, 0)

        scalar_phase(NCH - 1)
        pltpu.make_async_copy(h_hbm.at[src_v.at[0]], rows_v, sem).wait()
        scale_rows(rows_v)
        scatter(rows_v, NCH - 1)

        plsc.subcore_barrier()
        pltpu.sync_copy(num_sh.at[rsd], num_out.at[c].at[m].at[rsd])
        pltpu.sync_copy(den_sh.at[rsd], den_out.at[c].at[m].at[rsd])


def _sc_l1(h0, h1, es, ed, src, dst):
    mesh = plsc.VectorSubcoreMesh(core_axis_name="c", subcore_axis_name="s")
    f = functools.partial(
        pl.kernel, _sc_l1_body, mesh=mesh,
        compiler_params=pltpu.CompilerParams(needs_layout_passes=False, use_tc_tiling_on_sc=False),
        out_type=[jax.ShapeDtypeStruct((NC, 2, NP_PAD, HID), _f32),
                  jax.ShapeDtypeStruct((NC, 2, NP_PAD), _f32)],
        scratch_types=[
            pltpu.VMEM((N,), _f32),
            pltpu.VMEM((N,), _f32),
            pltpu.VMEM((NCH, CHUNK), _i32),
            pltpu.VMEM((NCH, CHUNK), _i32),
            pltpu.VMEM((CHUNK,), _f32),
            pltpu.VMEM((CHUNK, HID), _f32),
            pltpu.VMEM((CHUNK, HID), _f32),
            pltpu.VMEM((128, HID), _f32),
            pltpu.VMEM((640,), _f32),
            pltpu.VMEM_SHARED((NP_PAD, HID), _f32),
            pltpu.VMEM_SHARED((NP_PAD,), _f32),
            pltpu.SemaphoreType.DMA,
            pltpu.SemaphoreType.DMA,
        ])()
    return f(h0, h1, es, ed, src, dst)


def _sc_l2_body(h2_hbm, prm_hbm, src_hbm, dst_hbm,
                n0_out, n1_out, den_out,
                es_v, ed_v, h2_v, src_v, dst_v, w_v, m0_v, m1_v, zden_v,
                prm_v,
                n0a_sh, n0b_sh, n1a_sh, n1b_sh, dena_sh, denb_sh):
    c = lax.axis_index("c")
    s = lax.axis_index("s")
    wid = c * NS + s

    def zd(i, carry):
        zden_v[pl.ds(i * 16, 16)] = jnp.zeros((16,), _f32)
        return carry
    lax.fori_loop(0, 40, zd, 0)
    for sh in (n0a_sh, n0b_sh, n1a_sh, n1b_sh, dena_sh, denb_sh):
        pltpu.sync_copy(zden_v, sh.at[pl.ds(s * DPT, DPT)])
    plsc.subcore_barrier()

    z16 = jnp.zeros((16,), _i32)
    iot = lax.iota(_i32, 16)
    for m, (n0_sh, n1_sh, den_sh) in enumerate(
            ((n0a_sh, n1a_sh, dena_sh), (n0b_sh, n1b_sh, denb_sh))):
        pltpu.sync_copy(h2_hbm.at[m], h2_v)
        pltpu.sync_copy(prm_hbm.at[m], prm_v)
        pltpu.sync_copy(src_hbm.at[m].at[wid], src_v)
        pltpu.sync_copy(dst_hbm.at[m].at[wid], dst_v)
        as0 = plsc.load_gather(prm_v, [z16 + 1])
        as1 = plsc.load_gather(prm_v, [z16 + 2])
        ad0 = plsc.load_gather(prm_v, [z16 + 3])
        ad1 = plsc.load_gather(prm_v, [z16 + 4])

        def node_body(i, carry):
            idx16 = iot + i * 16
            c0 = plsc.load_gather(h2_v, [idx16, z16])
            c1 = plsc.load_gather(h2_v, [idx16, z16 + 1])
            sl = pl.ds(i * 16, 16)
            es_v[sl] = as0 * c0 + as1 * c1
            ed_v[sl] = ad0 * c0 + ad1 * c1
            return carry
        lax.fori_loop(0, N // 16, node_body, 0)

        def chunk_body(ci, carry):
            for i in range(CHUNK // 16):
                sl = pl.ds(i * 16, 16)
                s16 = src_v[ci, sl]
                d16 = dst_v[ci, sl]
                a = (plsc.load_gather(es_v, [s16])
                     + plsc.load_gather(ed_v, [d16]))
                a = jnp.where(a >= 0, a, 0.2 * a)
                w16 = jnp.exp(a)
                w_v[sl] = w16
                m0_v[sl] = w16 * plsc.load_gather(h2_v, [s16, z16])
                m1_v[sl] = w16 * plsc.load_gather(h2_v, [s16, z16 + 1])
            pltpu.sync_copy(w_v, den_sh.at[dst_v.at[ci]], add=True)
            pltpu.sync_copy(m0_v, n0_sh.at[dst_v.at[ci]], add=True)
            pltpu.sync_copy(m1_v, n1_sh.at[dst_v.at[ci]], add=True)
            return carry
        lax.fori_loop(0, NCH, chunk_body, 0)

    plsc.subcore_barrier()
    rs = pl.ds(s * DPT, DPT)
    pltpu.sync_copy(n0a_sh.at[rs], n0_out.at[c].at[0].at[rs])
    pltpu.sync_copy(n0b_sh.at[rs], n0_out.at[c].at[1].at[rs])
    pltpu.sync_copy(n1a_sh.at[rs], n1_out.at[c].at[0].at[rs])
    pltpu.sync_copy(n1b_sh.at[rs], n1_out.at[c].at[1].at[rs])
    pltpu.sync_copy(dena_sh.at[rs], den_out.at[c].at[0].at[rs])
    pltpu.sync_copy(denb_sh.at[rs], den_out.at[c].at[1].at[rs])


def _sc_l2(h2, prm, src, dst):
    mesh = plsc.VectorSubcoreMesh(core_axis_name="c", subcore_axis_name="s")
    f = functools.partial(
        pl.kernel, _sc_l2_body, mesh=mesh,
        compiler_params=pltpu.CompilerParams(needs_layout_passes=False, use_tc_tiling_on_sc=False),
        out_type=[jax.ShapeDtypeStruct((NC, 2, NP_PAD), _f32),
                  jax.ShapeDtypeStruct((NC, 2, NP_PAD), _f32),
                  jax.ShapeDtypeStruct((NC, 2, NP_PAD), _f32)],
        scratch_types=[
            pltpu.VMEM((N,), _f32),
            pltpu.VMEM((N,), _f32),
            pltpu.VMEM((N, 2), _f32),
            pltpu.VMEM((NCH, CHUNK), _i32),
            pltpu.VMEM((NCH, CHUNK), _i32),
            pltpu.VMEM((CHUNK,), _f32),
            pltpu.VMEM((CHUNK,), _f32),
            pltpu.VMEM((CHUNK,), _f32),
            pltpu.VMEM((640,), _f32),
            pltpu.VMEM((8,), _f32),
            pltpu.VMEM_SHARED((NP_PAD,), _f32),
            pltpu.VMEM_SHARED((NP_PAD,), _f32),
            pltpu.VMEM_SHARED((NP_PAD,), _f32),
            pltpu.VMEM_SHARED((NP_PAD,), _f32),
            pltpu.VMEM_SHARED((NP_PAD,), _f32),
            pltpu.VMEM_SHARED((NP_PAD,), _f32),
        ])()
    return f(h2, prm, src, dst)


def _tca_body(x_ref, W1_ref, as_ref, ad_ref,
              h0_ref, h1_ref, es0_ref, es1_ref, ed0_ref, ed1_ref):
    x = x_ref[...]
    outs = ((h0_ref, es0_ref, ed0_ref), (h1_ref, es1_ref, ed1_ref))
    for m in range(2):
        h = jnp.dot(x, W1_ref[m], preferred_element_type=_f32)
        h_ref, es_ref, ed_ref = outs[m]
        h_ref[...] = h
        es_ref[...] = jnp.dot(h, as_ref[m], preferred_element_type=_f32)
        ed_ref[...] = jnp.dot(h, ad_ref[m], preferred_element_type=_f32)


def _tca(x, W1, a_src1, a_dst1):
    blk = 1000
    grid = N // blk
    outs = [jax.ShapeDtypeStruct((N, HID), _f32)] * 2 + \
           [jax.ShapeDtypeStruct((N, 1), _f32)] * 4
    return pl.pallas_call(
        _tca_body,
        grid=(grid,),
        in_specs=[
            pl.BlockSpec((blk, D), lambda i: (i, 0)),
            pl.BlockSpec((2, D, HID), lambda i: (0, 0, 0)),
            pl.BlockSpec((2, HID, 1), lambda i: (0, 0, 0)),
            pl.BlockSpec((2, HID, 1), lambda i: (0, 0, 0)),
        ],
        out_specs=[pl.BlockSpec((blk, HID), lambda i: (i, 0))] * 2
                  + [pl.BlockSpec((blk, 1), lambda i: (i, 0))] * 4,
        out_shape=outs,
    )(x, W1, a_src1.reshape(2, HID, 1), a_dst1.reshape(2, HID, 1))


def _tcb_body(num_ref, den_ref, b1_ref, g_ref, be_ref, W2_ref, h2_ref):
    num = num_ref[0, 0] + num_ref[1, 0]
    den = den_ref[0, 0, 0] + den_ref[1, 0, 0]
    dcol = den.reshape(N, 1)
    hid = num / (dcol + 1e-16) + b1_ref[0]
    mean = jnp.mean(hid, axis=0, keepdims=True)
    var = jnp.mean((hid - mean) ** 2, axis=0, keepdims=True)
    hid = (hid - mean) / jnp.sqrt(var + 1e-5) * g_ref[0] + be_ref[0]
    hid = _leaky(hid, 0.01)
    h2_ref[0] = jnp.dot(hid, W2_ref[0], preferred_element_type=_f32)


def _tcb(num1, den1, b1, gamma, beta, W2):
    return pl.pallas_call(
        _tcb_body,
        grid=(2,),
        in_specs=[
            pl.BlockSpec((NC, 1, N, HID), lambda m: (0, m, 0, 0)),
            pl.BlockSpec((NC, 1, 1, N), lambda m: (0, m, 0, 0)),
            pl.BlockSpec((1, 1, HID), lambda m: (m, 0, 0)),
            pl.BlockSpec((1, 1, HID), lambda m: (m, 0, 0)),
            pl.BlockSpec((1, 1, HID), lambda m: (m, 0, 0)),
            pl.BlockSpec((1, HID, 2), lambda m: (m, 0, 0)),
        ],
        out_specs=pl.BlockSpec((1, N, 2), lambda m: (m, 0, 0)),
        out_shape=jax.ShapeDtypeStruct((2, N, 2), _f32),
    )(num1, den1.reshape(NC, 2, 1, N), b1.reshape(2, 1, HID),
      gamma.reshape(2, 1, HID), beta.reshape(2, 1, HID), W2)


def _tcc_body(n0_ref, n1_ref, den_ref, b2_ref, y_ref, mask_ref,
              np_ref, loss_ref):
    y = y_ref[...]
    maskf = mask_ref[...]
    msum = jnp.sum(maskf)
    np_acc = jnp.zeros((N,), _f32)
    loss_acc = jnp.zeros((), _f32)
    for m in range(2):
        den = den_ref[0, m] + den_ref[1, m]
        o0 = (n0_ref[0, m] + n0_ref[1, m]) / (den + 1e-16) + b2_ref[m, 0]
        o1 = (n1_ref[0, m] + n1_ref[1, m]) / (den + 1e-16) + b2_ref[m, 1]
        mx = jnp.maximum(o0, o1)
        l0 = o0 - mx
        l1 = o1 - mx
        lse = jnp.log(jnp.exp(l0) + jnp.exp(l1))
        lsm0 = l0 - lse
        lsm1 = l1 - lse
        np_acc = np_acc + jnp.exp(lsm1)
        picked = jnp.where(y == 1, lsm1, lsm0)
        loss_acc = loss_acc - jnp.sum(picked * maskf) / msum
    np_ref[...] = np_acc * 0.5
    loss_ref[...] = jnp.broadcast_to(loss_acc * 0.5, (1, 1))


def _tcc(n0, n1, den2, b2, y, maskf):
    full = lambda *shape: pl.BlockSpec(shape, lambda: tuple(0 for _ in shape))
    return pl.pallas_call(
        _tcc_body,
        in_specs=[full(NC, 2, N), full(NC, 2, N), full(NC, 2, N),
                  full(2, 2), full(N), full(N)],
        out_specs=[full(N), full(1, 1)],
        out_shape=[jax.ShapeDtypeStruct((N,), _f32),
                   jax.ShapeDtypeStruct((1, 1), _f32)],
    )(n0, n1, den2, b2, y, maskf)


def _tcd_body(x_ref, lp_ref, ew_ref, np_ref,
              norm_ref, cs_ref, pw_ref):
    lp = lp_ref[...]
    dp = _sig(lp)
    approx = (jnp.log(dp + EPS) - jnp.log(1.0 - dp + EPS)
              + jnp.log(0.5 + EPS) - jnp.log(1.0 - 0.5 + EPS))
    ao = _sig(approx / 0.1)
    ew = _sig(ew_ref[...])
    cs = (1.0 - ao) * ew
    A = x_ref[...] * cs
    rowsum = jnp.sum(A, axis=1, keepdims=True)
    v = (2.0 * np_ref[...] - 1.0) / (rowsum + 1e-6)
    t = jnp.sum(A * v, axis=0, keepdims=True)
    diff = jnp.sum(A * t, axis=1, keepdims=True)
    norm_ref[...] = _sig(diff)
    cs_ref[...] = cs
    pw_ref[...] = 1.0 - dp


def _tcd(x, logit_p, embedding_w, np_col):
    full = lambda *shape: pl.BlockSpec(shape, lambda: tuple(0 for _ in shape))
    return pl.pallas_call(
        _tcd_body,
        in_specs=[full(N, D), full(1, D), full(1, D), full(N, 1)],
        out_specs=[full(N, 1), full(1, D), full(1, D)],
        out_shape=[jax.ShapeDtypeStruct((N, 1), _f32),
                   jax.ShapeDtypeStruct((1, D), _f32),
                   jax.ShapeDtypeStruct((1, D), _f32)],
    )(x, logit_p.reshape(1, D), embedding_w.reshape(1, D), np_col)


def _gram_body(colscale_ref, xb_ref, xc_ref, out_ref):
    colscale = colscale_ref[...]
    xb = xb_ref[...] * colscale
    xc = xc_ref[...] * colscale
    ones = jnp.ones((1, D), _f32)
    rowsumc = jax.lax.dot_general(ones, xc, (((1,), (1,)), ((), ())),
                                  preferred_element_type=_f32)
    gram = jax.lax.dot_general(xb, xc, (((1,), (1,)), ((), ())),
                               preferred_element_type=_f32)
    out_ref[...] = gram / (rowsumc + 1e-6)


def _gram_pallas(x, colscale):
    gr = (N + ROW_BLK - 1) // ROW_BLK
    gc = (N + COL_BLK - 1) // COL_BLK
    return pl.pallas_call(
        _gram_body,
        grid=(gr, gc),
        in_specs=[
            pl.BlockSpec((1, D), lambda i, j: (0, 0)),
            pl.BlockSpec((ROW_BLK, D), lambda i, j: (i, 0)),
            pl.BlockSpec((COL_BLK, D), lambda i, j: (j, 0)),
        ],
        out_specs=pl.BlockSpec((ROW_BLK, COL_BLK), lambda i, j: (i, j)),
        out_shape=jax.ShapeDtypeStruct((N, N), _f32),
    )(colscale, x, x)


def kernel(x, W1, a_src1, a_dst1, b1, gamma, beta, W2, a_src2, a_dst2, b2,
           embedding_w, logit_p, edge_index_ppi, edge_index_homolog, y,
           train_mask):
    src = jnp.stack([edge_index_ppi[0].reshape(NW, NCH, CHUNK),
                     edge_index_homolog[0].reshape(NW, NCH, CHUNK)])
    dst = jnp.stack([edge_index_ppi[1].reshape(NW, NCH, CHUNK),
                     edge_index_homolog[1].reshape(NW, NCH, CHUNK)])

    h0, h1, es0, es1, ed0, ed1 = _tca(x, W1, a_src1, a_dst1)
    es1c = jnp.stack([es0.reshape(N), es1.reshape(N)])
    ed1c = jnp.stack([ed0.reshape(N), ed1.reshape(N)])

    num1p, den1p = _sc_l1(h0, h1, es1c, ed1c, src, dst)
    num1 = num1p[:, :, :N]
    den1 = den1p[:, :, :N]

    h2 = _tcb(num1, den1, b1, gamma, beta, W2)
    prm = jnp.concatenate(
        [jnp.zeros((2, 1), _f32), a_src2, a_dst2,
         jnp.zeros((2, 3), _f32)], axis=1)

    n0p, n1p, den2p = _sc_l2(h2, prm, src, dst)
    n0, n1, den2 = n0p[:, :, :N], n1p[:, :, :N], den2p[:, :, :N]

    node_p, loss2d = _tcc(n0, n1, den2, b2, y,
                          train_mask.astype(_f32))
    loss = loss2d.reshape(1)

    norm_col, colscale, pw2d = _tcd(x, logit_p, embedding_w,
                                    node_p.reshape(N, 1))
    norm1 = norm_col.reshape(N)
    pw_vimp = pw2d.reshape(D)

    graph = _gram_pallas(x, colscale)

    return (node_p, loss, norm1, graph, pw_vimp)


# speedup vs baseline: 39.1735x; 1.0377x over previous
import functools
import jax
import jax.numpy as jnp
import numpy as np
from jax import lax
from jax.experimental import pallas as pl
from jax.experimental.pallas import tpu as pltpu
from jax.experimental.pallas import tpu_sc as plsc

EPS = float(np.finfo(float).eps)
N = 10000
D = 128
HID = 64
E = 320000

NC = 2
NS = 16
NW = NC * NS
EPT = E // NW
CHUNK = 80
NCH = EPT // CHUNK
NPT = N // NS
NP_PAD = 10240
DPT = NP_PAD // NS

ROW_BLK = 1024
COL_BLK = 1024

_f32 = jnp.float32
_i32 = jnp.int32


def _leaky(x, slope):
    return jnp.where(x >= 0, x, slope * x)


def _sig(z):
    return 1.0 / (1.0 + jnp.exp(-z))


def _sc_l1_body(h0_hbm, h1_hbm, es_hbm, ed_hbm, src_hbm, dst_hbm,
                num_out, den_out,
                es_v, ed_v, src_v, dst_v, w_a, w_b, rows_v, rows_b, znum_v,
                zden_v, idx0_v, num_sh, den_sh,
                sem, sem_b, sna, snb, sda, sdb):
    c = lax.axis_index("c")
    s = lax.axis_index("s")
    wid = c * NS + s

    def zrow(r, carry):
        for q in range(4):
            znum_v[r, pl.ds(q * 16, 16)] = jnp.zeros((16,), _f32)
        return carry
    lax.fori_loop(0, 128, zrow, 0)

    def zd(i, carry):
        zden_v[pl.ds(i * 16, 16)] = jnp.zeros((16,), _f32)
        return carry
    lax.fori_loop(0, 40, zd, 0)
    for i in range(CHUNK // 16):
        idx0_v[pl.ds(i * 16, 16)] = jnp.zeros((16,), _i32) + N

    pltpu.async_copy(znum_v.at[pl.ds(0, CHUNK)], num_sh.at[idx0_v], snb,
                     add=True)
    pltpu.async_copy(zden_v.at[pl.ds(0, CHUNK)], den_sh.at[idx0_v], sdb,
                     add=True)

    def wait_num(semref, rows):
        pltpu.make_async_copy(rows, num_sh.at[idx0_v], semref).wait()

    def wait_den(semref, wbuf):
        pltpu.make_async_copy(wbuf, den_sh.at[idx0_v], semref).wait()

    def wait_gather(semref, rows, h_hbm):
        pltpu.make_async_copy(h_hbm.at[src_v.at[0]], rows, semref).wait()

    rsd = pl.ds(s * DPT, DPT)
    for m, h_hbm in enumerate((h0_hbm, h1_hbm)):
        for j in range(5):
            pltpu.sync_copy(znum_v, num_sh.at[pl.ds(s * DPT + j * 128, 128)])
        pltpu.sync_copy(zden_v, den_sh.at[rsd])
        plsc.subcore_barrier()

        pltpu.sync_copy(es_hbm.at[m], es_v)
        pltpu.sync_copy(ed_hbm.at[m], ed_v)
        pltpu.sync_copy(src_hbm.at[m].at[wid], src_v)
        pltpu.sync_copy(dst_hbm.at[m].at[wid], dst_v)

        def scalar_phase(ci, wbuf):
            for i in range(CHUNK // 16):
                sl = pl.ds(i * 16, 16)
                s16 = src_v[ci, sl]
                d16 = dst_v[ci, sl]
                a = (plsc.load_gather(es_v, [s16])
                     + plsc.load_gather(ed_v, [d16]))
                a = jnp.where(a >= 0, a, 0.2 * a)
                wbuf[sl] = jnp.exp(a)

        def scale_rows(rows, wbuf):
            def scale_g(g, carry2):
                base = g * 16
                w16 = wbuf[pl.ds(base, 16)]
                for j in range(16):
                    ws = lax.gather(
                        w16, jnp.full((16, 1), j, _i32),
                        lax.GatherDimensionNumbers(
                            offset_dims=(), collapsed_slice_dims=(0,),
                            start_index_map=(0,)),
                        (1,), mode=lax.GatherScatterMode.PROMISE_IN_BOUNDS)
                    e = base + j
                    for q in range(HID // 16):
                        sl = pl.ds(q * 16, 16)
                        rows[e, sl] = rows[e, sl] * ws
                return carry2
            lax.fori_loop(0, CHUNK // 16, scale_g, 0)

        def scatter(rows, wbuf, ci, sn, sd):
            pltpu.async_copy(rows, num_sh.at[dst_v.at[ci]], sn, add=True)
            pltpu.async_copy(wbuf, den_sh.at[dst_v.at[ci]], sd, add=True)

        cp0 = pltpu.async_copy(h_hbm.at[src_v.at[0]], rows_v, sem)

        def pair_body(k, carry):
            ca = 2 * k
            cb = 2 * k + 1
            scalar_phase(ca, w_a)
            wait_num(snb, rows_b)
            wait_den(sdb, w_b)
            pltpu.async_copy(h_hbm.at[src_v.at[cb]], rows_b, sem_b)
            wait_gather(sem, rows_v, h_hbm)
            scale_rows(rows_v, w_a)
            scatter(rows_v, w_a, ca, sna, sda)
            scalar_phase(cb, w_b)
            wait_num(sna, rows_v)
            wait_den(sda, w_a)
            pltpu.async_copy(h_hbm.at[src_v.at[cb + 1]], rows_v, sem)
            wait_gather(sem_b, rows_b, h_hbm)
            scale_rows(rows_b, w_b)
            scatter(rows_b, w_b, cb, snb, sdb)
            return carry
        lax.fori_loop(0, (NCH - 1) // 2, pair_body, 0)

        scalar_phase(NCH - 1, w_a)
        wait_gather(sem, rows_v, h_hbm)
        scale_rows(rows_v, w_a)
        scatter(rows_v, w_a, NCH - 1, sna, sda)
        wait_num(sna, rows_v)
        wait_den(sda, w_a)
        wait_num(snb, rows_b)
        wait_den(sdb, w_b)
        if m == 0:
            pltpu.async_copy(znum_v.at[pl.ds(0, CHUNK)], num_sh.at[idx0_v],
                             snb, add=True)
            pltpu.async_copy(zden_v.at[pl.ds(0, CHUNK)], den_sh.at[idx0_v],
                             sdb, add=True)
        plsc.subcore_barrier()
        pltpu.sync_copy(num_sh.at[rsd], num_out.at[c].at[m].at[rsd])
        pltpu.sync_copy(den_sh.at[rsd], den_out.at[c].at[m].at[rsd])
        plsc.subcore_barrier()


def _sc_l1(h0, h1, es, ed, src, dst):
    mesh = plsc.VectorSubcoreMesh(core_axis_name="c", subcore_axis_name="s")
    f = functools.partial(
        pl.kernel, _sc_l1_body, mesh=mesh,
        compiler_params=pltpu.CompilerParams(needs_layout_passes=False, use_tc_tiling_on_sc=False),
        out_type=[jax.ShapeDtypeStruct((NC, 2, NP_PAD, HID), _f32),
                  jax.ShapeDtypeStruct((NC, 2, NP_PAD), _f32)],
        scratch_types=[
            pltpu.VMEM((N,), _f32),
            pltpu.VMEM((N,), _f32),
            pltpu.VMEM((NCH, CHUNK), _i32),
            pltpu.VMEM((NCH, CHUNK), _i32),
            pltpu.VMEM((CHUNK,), _f32),
            pltpu.VMEM((CHUNK,), _f32),
            pltpu.VMEM((CHUNK, HID), _f32),
            pltpu.VMEM((CHUNK, HID), _f32),
            pltpu.VMEM((128, HID), _f32),
            pltpu.VMEM((640,), _f32),
            pltpu.VMEM((CHUNK,), _i32),
            pltpu.VMEM_SHARED((NP_PAD, HID), _f32),
            pltpu.VMEM_SHARED((NP_PAD,), _f32),
            pltpu.SemaphoreType.DMA,
            pltpu.SemaphoreType.DMA,
            pltpu.SemaphoreType.DMA,
            pltpu.SemaphoreType.DMA,
            pltpu.SemaphoreType.DMA,
            pltpu.SemaphoreType.DMA,
        ])()
    return f(h0, h1, es, ed, src, dst)


def _sc_l2_body(h2_hbm, prm_hbm, src_hbm, dst_hbm,
                n0_out, n1_out, den_out,
                es_v, ed_v, h2_v, src_v, dst_v, w_v, m0_v, m1_v, zden_v,
                prm_v,
                n0a_sh, n0b_sh, n1a_sh, n1b_sh, dena_sh, denb_sh):
    c = lax.axis_index("c")
    s = lax.axis_index("s")
    wid = c * NS + s

    def zd(i, carry):
        zden_v[pl.ds(i * 16, 16)] = jnp.zeros((16,), _f32)
        return carry
    lax.fori_loop(0, 40, zd, 0)
    for sh in (n0a_sh, n0b_sh, n1a_sh, n1b_sh, dena_sh, denb_sh):
        pltpu.sync_copy(zden_v, sh.at[pl.ds(s * DPT, DPT)])
    plsc.subcore_barrier()

    z16 = jnp.zeros((16,), _i32)
    iot = lax.iota(_i32, 16)
    for m, (n0_sh, n1_sh, den_sh) in enumerate(
            ((n0a_sh, n1a_sh, dena_sh), (n0b_sh, n1b_sh, denb_sh))):
        pltpu.sync_copy(h2_hbm.at[m], h2_v)
        pltpu.sync_copy(prm_hbm.at[m], prm_v)
        pltpu.sync_copy(src_hbm.at[m].at[wid], src_v)
        pltpu.sync_copy(dst_hbm.at[m].at[wid], dst_v)
        as0 = plsc.load_gather(prm_v, [z16 + 1])
        as1 = plsc.load_gather(prm_v, [z16 + 2])
        ad0 = plsc.load_gather(prm_v, [z16 + 3])
        ad1 = plsc.load_gather(prm_v, [z16 + 4])

        def node_body(i, carry):
            idx16 = iot + i * 16
            c0 = plsc.load_gather(h2_v, [idx16, z16])
            c1 = plsc.load_gather(h2_v, [idx16, z16 + 1])
            sl = pl.ds(i * 16, 16)
            es_v[sl] = as0 * c0 + as1 * c1
            ed_v[sl] = ad0 * c0 + ad1 * c1
            return carry
        lax.fori_loop(0, N // 16, node_body, 0)

        def chunk_body(ci, carry):
            for i in range(CHUNK // 16):
                sl = pl.ds(i * 16, 16)
                s16 = src_v[ci, sl]
                d16 = dst_v[ci, sl]
                a = (plsc.load_gather(es_v, [s16])
                     + plsc.load_gather(ed_v, [d16]))
                a = jnp.where(a >= 0, a, 0.2 * a)
                w16 = jnp.exp(a)
                w_v[sl] = w16
                m0_v[sl] = w16 * plsc.load_gather(h2_v, [s16, z16])
                m1_v[sl] = w16 * plsc.load_gather(h2_v, [s16, z16 + 1])
            pltpu.sync_copy(w_v, den_sh.at[dst_v.at[ci]], add=True)
            pltpu.sync_copy(m0_v, n0_sh.at[dst_v.at[ci]], add=True)
            pltpu.sync_copy(m1_v, n1_sh.at[dst_v.at[ci]], add=True)
            return carry
        lax.fori_loop(0, NCH, chunk_body, 0)

    plsc.subcore_barrier()
    rs = pl.ds(s * DPT, DPT)
    pltpu.sync_copy(n0a_sh.at[rs], n0_out.at[c].at[0].at[rs])
    pltpu.sync_copy(n0b_sh.at[rs], n0_out.at[c].at[1].at[rs])
    pltpu.sync_copy(n1a_sh.at[rs], n1_out.at[c].at[0].at[rs])
    pltpu.sync_copy(n1b_sh.at[rs], n1_out.at[c].at[1].at[rs])
    pltpu.sync_copy(dena_sh.at[rs], den_out.at[c].at[0].at[rs])
    pltpu.sync_copy(denb_sh.at[rs], den_out.at[c].at[1].at[rs])


def _sc_l2(h2, prm, src, dst):
    mesh = plsc.VectorSubcoreMesh(core_axis_name="c", subcore_axis_name="s")
    f = functools.partial(
        pl.kernel, _sc_l2_body, mesh=mesh,
        compiler_params=pltpu.CompilerParams(needs_layout_passes=False, use_tc_tiling_on_sc=False),
        out_type=[jax.ShapeDtypeStruct((NC, 2, NP_PAD), _f32),
                  jax.ShapeDtypeStruct((NC, 2, NP_PAD), _f32),
                  jax.ShapeDtypeStruct((NC, 2, NP_PAD), _f32)],
        scratch_types=[
            pltpu.VMEM((N,), _f32),
            pltpu.VMEM((N,), _f32),
            pltpu.VMEM((N, 2), _f32),
            pltpu.VMEM((NCH, CHUNK), _i32),
            pltpu.VMEM((NCH, CHUNK), _i32),
            pltpu.VMEM((CHUNK,), _f32),
            pltpu.VMEM((CHUNK,), _f32),
            pltpu.VMEM((CHUNK,), _f32),
            pltpu.VMEM((640,), _f32),
            pltpu.VMEM((8,), _f32),
            pltpu.VMEM_SHARED((NP_PAD,), _f32),
            pltpu.VMEM_SHARED((NP_PAD,), _f32),
            pltpu.VMEM_SHARED((NP_PAD,), _f32),
            pltpu.VMEM_SHARED((NP_PAD,), _f32),
            pltpu.VMEM_SHARED((NP_PAD,), _f32),
            pltpu.VMEM_SHARED((NP_PAD,), _f32),
        ])()
    return f(h2, prm, src, dst)


def _tca_body(x_ref, W1_ref, as_ref, ad_ref,
              h0_ref, h1_ref, es0_ref, es1_ref, ed0_ref, ed1_ref):
    x = x_ref[...]
    outs = ((h0_ref, es0_ref, ed0_ref), (h1_ref, es1_ref, ed1_ref))
    for m in range(2):
        h = jnp.dot(x, W1_ref[m], preferred_element_type=_f32)
        h_ref, es_ref, ed_ref = outs[m]
        h_ref[...] = h
        es_ref[...] = jnp.dot(h, as_ref[m], preferred_element_type=_f32)
        ed_ref[...] = jnp.dot(h, ad_ref[m], preferred_element_type=_f32)


def _tca(x, W1, a_src1, a_dst1):
    blk = 1000
    grid = N // blk
    outs = [jax.ShapeDtypeStruct((N, HID), _f32)] * 2 + \
           [jax.ShapeDtypeStruct((N, 1), _f32)] * 4
    return pl.pallas_call(
        _tca_body,
        grid=(grid,),
        in_specs=[
            pl.BlockSpec((blk, D), lambda i: (i, 0)),
            pl.BlockSpec((2, D, HID), lambda i: (0, 0, 0)),
            pl.BlockSpec((2, HID, 1), lambda i: (0, 0, 0)),
            pl.BlockSpec((2, HID, 1), lambda i: (0, 0, 0)),
        ],
        out_specs=[pl.BlockSpec((blk, HID), lambda i: (i, 0))] * 2
                  + [pl.BlockSpec((blk, 1), lambda i: (i, 0))] * 4,
        out_shape=outs,
    )(x, W1, a_src1.reshape(2, HID, 1), a_dst1.reshape(2, HID, 1))


def _tcb_body(num_ref, den_ref, b1_ref, g_ref, be_ref, W2_ref, h2_ref):
    num = num_ref[0, 0] + num_ref[1, 0]
    den = den_ref[0, 0, 0] + den_ref[1, 0, 0]
    dcol = den.reshape(N, 1)
    hid = num / (dcol + 1e-16) + b1_ref[0]
    mean = jnp.mean(hid, axis=0, keepdims=True)
    var = jnp.mean((hid - mean) ** 2, axis=0, keepdims=True)
    hid = (hid - mean) / jnp.sqrt(var + 1e-5) * g_ref[0] + be_ref[0]
    hid = _leaky(hid, 0.01)
    h2_ref[0] = jnp.dot(hid, W2_ref[0], preferred_element_type=_f32)


def _tcb(num1, den1, b1, gamma, beta, W2):
    return pl.pallas_call(
        _tcb_body,
        grid=(2,),
        in_specs=[
            pl.BlockSpec((NC, 1, N, HID), lambda m: (0, m, 0, 0)),
            pl.BlockSpec((NC, 1, 1, N), lambda m: (0, m, 0, 0)),
            pl.BlockSpec((1, 1, HID), lambda m: (m, 0, 0)),
            pl.BlockSpec((1, 1, HID), lambda m: (m, 0, 0)),
            pl.BlockSpec((1, 1, HID), lambda m: (m, 0, 0)),
            pl.BlockSpec((1, HID, 2), lambda m: (m, 0, 0)),
        ],
        out_specs=pl.BlockSpec((1, N, 2), lambda m: (m, 0, 0)),
        out_shape=jax.ShapeDtypeStruct((2, N, 2), _f32),
    )(num1, den1.reshape(NC, 2, 1, N), b1.reshape(2, 1, HID),
      gamma.reshape(2, 1, HID), beta.reshape(2, 1, HID), W2)


def _tcc_body(n0_ref, n1_ref, den_ref, b2_ref, y_ref, mask_ref,
              np_ref, loss_ref):
    y = y_ref[...]
    maskf = mask_ref[...]
    msum = jnp.sum(maskf)
    np_acc = jnp.zeros((N,), _f32)
    loss_acc = jnp.zeros((), _f32)
    for m in range(2):
        den = den_ref[0, m] + den_ref[1, m]
        o0 = (n0_ref[0, m] + n0_ref[1, m]) / (den + 1e-16) + b2_ref[m, 0]
        o1 = (n1_ref[0, m] + n1_ref[1, m]) / (den + 1e-16) + b2_ref[m, 1]
        mx = jnp.maximum(o0, o1)
        l0 = o0 - mx
        l1 = o1 - mx
        lse = jnp.log(jnp.exp(l0) + jnp.exp(l1))
        lsm0 = l0 - lse
        lsm1 = l1 - lse
        np_acc = np_acc + jnp.exp(lsm1)
        picked = jnp.where(y == 1, lsm1, lsm0)
        loss_acc = loss_acc - jnp.sum(picked * maskf) / msum
    np_ref[...] = np_acc * 0.5
    loss_ref[...] = jnp.broadcast_to(loss_acc * 0.5, (1, 1))


def _tcc(n0, n1, den2, b2, y, maskf):
    full = lambda *shape: pl.BlockSpec(shape, lambda: tuple(0 for _ in shape))
    return pl.pallas_call(
        _tcc_body,
        in_specs=[full(NC, 2, N), full(NC, 2, N), full(NC, 2, N),
                  full(2, 2), full(N), full(N)],
        out_specs=[full(N), full(1, 1)],
        out_shape=[jax.ShapeDtypeStruct((N,), _f32),
                   jax.ShapeDtypeStruct((1, 1), _f32)],
    )(n0, n1, den2, b2, y, maskf)


def _tcd_body(x_ref, lp_ref, ew_ref, np_ref,
              norm_ref, cs_ref, pw_ref):
    lp = lp_ref[...]
    dp = _sig(lp)
    approx = (jnp.log(dp + EPS) - jnp.log(1.0 - dp + EPS)
              + jnp.log(0.5 + EPS) - jnp.log(1.0 - 0.5 + EPS))
    ao = _sig(approx / 0.1)
    ew = _sig(ew_ref[...])
    cs = (1.0 - ao) * ew
    A = x_ref[...] * cs
    rowsum = jnp.sum(A, axis=1, keepdims=True)
    v = (2.0 * np_ref[...] - 1.0) / (rowsum + 1e-6)
    t = jnp.sum(A * v, axis=0, keepdims=True)
    diff = jnp.sum(A * t, axis=1, keepdims=True)
    norm_ref[...] = _sig(diff)
    cs_ref[...] = cs
    pw_ref[...] = 1.0 - dp


def _tcd(x, logit_p, embedding_w, np_col):
    full = lambda *shape: pl.BlockSpec(shape, lambda: tuple(0 for _ in shape))
    return pl.pallas_call(
        _tcd_body,
        in_specs=[full(N, D), full(1, D), full(1, D), full(N, 1)],
        out_specs=[full(N, 1), full(1, D), full(1, D)],
        out_shape=[jax.ShapeDtypeStruct((N, 1), _f32),
                   jax.ShapeDtypeStruct((1, D), _f32),
                   jax.ShapeDtypeStruct((1, D), _f32)],
    )(x, logit_p.reshape(1, D), embedding_w.reshape(1, D), np_col)


def _gram_body(colscale_ref, xb_ref, xc_ref, out_ref):
    colscale = colscale_ref[...]
    xb = xb_ref[...] * colscale
    xc = xc_ref[...] * colscale
    ones = jnp.ones((1, D), _f32)
    rowsumc = jax.lax.dot_general(ones, xc, (((1,), (1,)), ((), ())),
                                  preferred_element_type=_f32)
    gram = jax.lax.dot_general(xb, xc, (((1,), (1,)), ((), ())),
                               preferred_element_type=_f32)
    out_ref[...] = gram / (rowsumc + 1e-6)


def _gram_pallas(x, colscale):
    gr = (N + ROW_BLK - 1) // ROW_BLK
    gc = (N + COL_BLK - 1) // COL_BLK
    return pl.pallas_call(
        _gram_body,
        grid=(gr, gc),
        in_specs=[
            pl.BlockSpec((1, D), lambda i, j: (0, 0)),
            pl.BlockSpec((ROW_BLK, D), lambda i, j: (i, 0)),
            pl.BlockSpec((COL_BLK, D), lambda i, j: (j, 0)),
        ],
        out_specs=pl.BlockSpec((ROW_BLK, COL_BLK), lambda i, j: (i, j)),
        out_shape=jax.ShapeDtypeStruct((N, N), _f32),
    )(colscale, x, x)


def kernel(x, W1, a_src1, a_dst1, b1, gamma, beta, W2, a_src2, a_dst2, b2,
           embedding_w, logit_p, edge_index_ppi, edge_index_homolog, y,
           train_mask):
    src = jnp.stack([edge_index_ppi[0].reshape(NW, NCH, CHUNK),
                     edge_index_homolog[0].reshape(NW, NCH, CHUNK)])
    dst = jnp.stack([edge_index_ppi[1].reshape(NW, NCH, CHUNK),
                     edge_index_homolog[1].reshape(NW, NCH, CHUNK)])

    h0, h1, es0, es1, ed0, ed1 = _tca(x, W1, a_src1, a_dst1)
    es1c = jnp.stack([es0.reshape(N), es1.reshape(N)])
    ed1c = jnp.stack([ed0.reshape(N), ed1.reshape(N)])

    num1p, den1p = _sc_l1(h0, h1, es1c, ed1c, src, dst)
    num1 = num1p[:, :, :N]
    den1 = den1p[:, :, :N]

    h2 = _tcb(num1, den1, b1, gamma, beta, W2)
    prm = jnp.concatenate(
        [jnp.zeros((2, 1), _f32), a_src2, a_dst2,
         jnp.zeros((2, 3), _f32)], axis=1)

    n0p, n1p, den2p = _sc_l2(h2, prm, src, dst)
    n0, n1, den2 = n0p[:, :, :N], n1p[:, :, :N], den2p[:, :, :N]

    node_p, loss2d = _tcc(n0, n1, den2, b2, y,
                          train_mask.astype(_f32))
    loss = loss2d.reshape(1)

    norm_col, colscale, pw2d = _tcd(x, logit_p, embedding_w,
                                    node_p.reshape(N, 1))
    norm1 = norm_col.reshape(N)
    pw_vimp = pw2d.reshape(D)

    graph = _gram_pallas(x, colscale)

    return (node_p, loss, norm1, graph, pw_vimp)


# speedup vs baseline: 39.7730x; 1.0153x over previous
import functools
import jax
import jax.numpy as jnp
import numpy as np
from jax import lax
from jax.experimental import pallas as pl
from jax.experimental.pallas import tpu as pltpu
from jax.experimental.pallas import tpu_sc as plsc

EPS = float(np.finfo(float).eps)
N = 10000
D = 128
HID = 64
E = 320000

NC = 2
NS = 16
NW = NC * NS
EPT = E // NW
CHUNK = 80
NCH = EPT // CHUNK
NPT = N // NS
NP_PAD = 10240
DPT = NP_PAD // NS

ROW_BLK = 1024
COL_BLK = 1024

_f32 = jnp.float32
_i32 = jnp.int32


def _leaky(x, slope):
    return jnp.where(x >= 0, x, slope * x)


def _sig(z):
    return 1.0 / (1.0 + jnp.exp(-z))


def _sc_l1_body(h0_hbm, h1_hbm, es_hbm, ed_hbm, src_hbm, dst_hbm,
                num_out, den_out,
                es_v, ed_v, src_v, dst_v, w_a, w_b, rows_v, rows_b, znum_v,
                zden_v, idx0_v, num_sh, den_sh,
                sem, sem_b, sna, snb, sda, sdb):
    c = lax.axis_index("c")
    s = lax.axis_index("s")
    wid = c * NS + s

    def zrow(r, carry):
        for q in range(4):
            znum_v[r, pl.ds(q * 16, 16)] = jnp.zeros((16,), _f32)
        return carry
    lax.fori_loop(0, 128, zrow, 0)

    def zd(i, carry):
        zden_v[pl.ds(i * 16, 16)] = jnp.zeros((16,), _f32)
        return carry
    lax.fori_loop(0, 40, zd, 0)
    for i in range(CHUNK // 16):
        idx0_v[pl.ds(i * 16, 16)] = jnp.zeros((16,), _i32) + N

    pltpu.async_copy(znum_v.at[pl.ds(0, CHUNK)], num_sh.at[idx0_v], snb,
                     add=True)
    pltpu.async_copy(zden_v.at[pl.ds(0, CHUNK)], den_sh.at[idx0_v], sdb,
                     add=True)

    def wait_num(semref, rows):
        pltpu.make_async_copy(rows, num_sh.at[idx0_v], semref).wait()

    def wait_den(semref, wbuf):
        pltpu.make_async_copy(wbuf, den_sh.at[idx0_v], semref).wait()

    def wait_gather(semref, rows, h_hbm):
        pltpu.make_async_copy(h_hbm.at[src_v.at[0]], rows, semref).wait()

    rsd = pl.ds(s * DPT, DPT)
    for m, h_hbm in enumerate((h0_hbm, h1_hbm)):
        for j in range(5):
            pltpu.sync_copy(znum_v, num_sh.at[pl.ds(s * DPT + j * 128, 128)])
        pltpu.sync_copy(zden_v, den_sh.at[rsd])
        plsc.subcore_barrier()

        pltpu.sync_copy(es_hbm.at[m], es_v)
        pltpu.sync_copy(ed_hbm.at[m], ed_v)
        pltpu.sync_copy(src_hbm.at[m].at[wid], src_v)
        pltpu.sync_copy(dst_hbm.at[m].at[wid], dst_v)

        def scalar_phase(ci, wbuf):
            for i in range(CHUNK // 16):
                sl = pl.ds(i * 16, 16)
                s16 = src_v[ci, sl]
                d16 = dst_v[ci, sl]
                a = (plsc.load_gather(es_v, [s16])
                     + plsc.load_gather(ed_v, [d16]))
                a = jnp.where(a >= 0, a, 0.2 * a)
                wbuf[sl] = jnp.exp(a)

        def scale_rows(rows, wbuf):
            def scale_g(g, carry2):
                base = g * 16
                w16 = wbuf[pl.ds(base, 16)]
                for j in range(16):
                    ws = lax.gather(
                        w16, jnp.full((16, 1), j, _i32),
                        lax.GatherDimensionNumbers(
                            offset_dims=(), collapsed_slice_dims=(0,),
                            start_index_map=(0,)),
                        (1,), mode=lax.GatherScatterMode.PROMISE_IN_BOUNDS)
                    e = base + j
                    for q in range(HID // 16):
                        sl = pl.ds(q * 16, 16)
                        rows[e, sl] = rows[e, sl] * ws
                return carry2
            lax.fori_loop(0, CHUNK // 16, scale_g, 0)

        def scatter(rows, wbuf, ci, sn, sd):
            pltpu.async_copy(rows, num_sh.at[dst_v.at[ci]], sn, add=True)
            pltpu.async_copy(wbuf, den_sh.at[dst_v.at[ci]], sd, add=True)

        cp0 = pltpu.async_copy(h_hbm.at[src_v.at[0]], rows_v, sem)

        def pair_body(k, carry):
            ca = 2 * k
            cb = 2 * k + 1
            scalar_phase(ca, w_a)
            wait_num(snb, rows_b)
            wait_den(sdb, w_b)
            pltpu.async_copy(h_hbm.at[src_v.at[cb]], rows_b, sem_b)
            wait_gather(sem, rows_v, h_hbm)
            scale_rows(rows_v, w_a)
            scatter(rows_v, w_a, ca, sna, sda)
            scalar_phase(cb, w_b)
            wait_num(sna, rows_v)
            wait_den(sda, w_a)
            pltpu.async_copy(h_hbm.at[src_v.at[cb + 1]], rows_v, sem)
            wait_gather(sem_b, rows_b, h_hbm)
            scale_rows(rows_b, w_b)
            scatter(rows_b, w_b, cb, snb, sdb)
            return carry
        lax.fori_loop(0, (NCH - 1) // 2, pair_body, 0)

        scalar_phase(NCH - 1, w_a)
        wait_gather(sem, rows_v, h_hbm)
        scale_rows(rows_v, w_a)
        scatter(rows_v, w_a, NCH - 1, sna, sda)
        wait_num(sna, rows_v)
        wait_den(sda, w_a)
        wait_num(snb, rows_b)
        wait_den(sdb, w_b)
        if m == 0:
            pltpu.async_copy(znum_v.at[pl.ds(0, CHUNK)], num_sh.at[idx0_v],
                             snb, add=True)
            pltpu.async_copy(zden_v.at[pl.ds(0, CHUNK)], den_sh.at[idx0_v],
                             sdb, add=True)
        plsc.subcore_barrier()
        pltpu.sync_copy(num_sh.at[rsd], num_out.at[c].at[m].at[rsd])
        pltpu.sync_copy(den_sh.at[rsd], den_out.at[c].at[m].at[rsd])
        plsc.subcore_barrier()


def _sc_l1(h0, h1, es, ed, src, dst):
    mesh = plsc.VectorSubcoreMesh(core_axis_name="c", subcore_axis_name="s")
    f = functools.partial(
        pl.kernel, _sc_l1_body, mesh=mesh,
        compiler_params=pltpu.CompilerParams(needs_layout_passes=False, use_tc_tiling_on_sc=False),
        out_type=[jax.ShapeDtypeStruct((NC, 2, NP_PAD, HID), _f32),
                  jax.ShapeDtypeStruct((NC, 2, NP_PAD), _f32)],
        scratch_types=[
            pltpu.VMEM((N,), _f32),
            pltpu.VMEM((N,), _f32),
            pltpu.VMEM((NCH, CHUNK), _i32),
            pltpu.VMEM((NCH, CHUNK), _i32),
            pltpu.VMEM((CHUNK,), _f32),
            pltpu.VMEM((CHUNK,), _f32),
            pltpu.VMEM((CHUNK, HID), _f32),
            pltpu.VMEM((CHUNK, HID), _f32),
            pltpu.VMEM((128, HID), _f32),
            pltpu.VMEM((640,), _f32),
            pltpu.VMEM((CHUNK,), _i32),
            pltpu.VMEM_SHARED((NP_PAD, HID), _f32),
            pltpu.VMEM_SHARED((NP_PAD,), _f32),
            pltpu.SemaphoreType.DMA,
            pltpu.SemaphoreType.DMA,
            pltpu.SemaphoreType.DMA,
            pltpu.SemaphoreType.DMA,
            pltpu.SemaphoreType.DMA,
            pltpu.SemaphoreType.DMA,
        ])()
    return f(h0, h1, es, ed, src, dst)


def _sc_l2_body(h2_hbm, prm_hbm, src_hbm, dst_hbm,
                n0_out, n1_out, den_out,
                es_v, ed_v, h2_v, src_v, dst_v, w_v, m0_v, m1_v, zden_v,
                prm_v,
                n0a_sh, n0b_sh, n1a_sh, n1b_sh, dena_sh, denb_sh):
    c = lax.axis_index("c")
    s = lax.axis_index("s")
    wid = c * NS + s

    def zd(i, carry):
        zden_v[pl.ds(i * 16, 16)] = jnp.zeros((16,), _f32)
        return carry
    lax.fori_loop(0, 40, zd, 0)
    for sh in (n0a_sh, n0b_sh, n1a_sh, n1b_sh, dena_sh, denb_sh):
        pltpu.sync_copy(zden_v, sh.at[pl.ds(s * DPT, DPT)])
    plsc.subcore_barrier()

    z16 = jnp.zeros((16,), _i32)
    iot = lax.iota(_i32, 16)
    for m, (n0_sh, n1_sh, den_sh) in enumerate(
            ((n0a_sh, n1a_sh, dena_sh), (n0b_sh, n1b_sh, denb_sh))):
        pltpu.sync_copy(h2_hbm.at[m], h2_v)
        pltpu.sync_copy(prm_hbm.at[m], prm_v)
        pltpu.sync_copy(src_hbm.at[m].at[wid], src_v)
        pltpu.sync_copy(dst_hbm.at[m].at[wid], dst_v)
        as0 = plsc.load_gather(prm_v, [z16 + 1])
        as1 = plsc.load_gather(prm_v, [z16 + 2])
        ad0 = plsc.load_gather(prm_v, [z16 + 3])
        ad1 = plsc.load_gather(prm_v, [z16 + 4])

        def node_body(i, carry):
            idx16 = iot + i * 16
            c0 = plsc.load_gather(h2_v, [idx16, z16])
            c1 = plsc.load_gather(h2_v, [idx16, z16 + 1])
            sl = pl.ds(i * 16, 16)
            es_v[sl] = as0 * c0 + as1 * c1
            ed_v[sl] = ad0 * c0 + ad1 * c1
            return carry
        lax.fori_loop(0, N // 16, node_body, 0)

        def chunk_body(ci, carry):
            for i in range(CHUNK // 16):
                sl = pl.ds(i * 16, 16)
                s16 = src_v[ci, sl]
                d16 = dst_v[ci, sl]
                a = (plsc.load_gather(es_v, [s16])
                     + plsc.load_gather(ed_v, [d16]))
                a = jnp.where(a >= 0, a, 0.2 * a)
                w16 = jnp.exp(a)
                w_v[sl] = w16
                m0_v[sl] = w16 * plsc.load_gather(h2_v, [s16, z16])
                m1_v[sl] = w16 * plsc.load_gather(h2_v, [s16, z16 + 1])
            pltpu.sync_copy(w_v, den_sh.at[dst_v.at[ci]], add=True)
            pltpu.sync_copy(m0_v, n0_sh.at[dst_v.at[ci]], add=True)
            pltpu.sync_copy(m1_v, n1_sh.at[dst_v.at[ci]], add=True)
            return carry
        lax.fori_loop(0, NCH, chunk_body, 0)

    plsc.subcore_barrier()
    rs = pl.ds(s * DPT, DPT)
    pltpu.sync_copy(n0a_sh.at[rs], n0_out.at[c].at[0].at[rs])
    pltpu.sync_copy(n0b_sh.at[rs], n0_out.at[c].at[1].at[rs])
    pltpu.sync_copy(n1a_sh.at[rs], n1_out.at[c].at[0].at[rs])
    pltpu.sync_copy(n1b_sh.at[rs], n1_out.at[c].at[1].at[rs])
    pltpu.sync_copy(dena_sh.at[rs], den_out.at[c].at[0].at[rs])
    pltpu.sync_copy(denb_sh.at[rs], den_out.at[c].at[1].at[rs])


def _sc_l2(h2, prm, src, dst):
    mesh = plsc.VectorSubcoreMesh(core_axis_name="c", subcore_axis_name="s")
    f = functools.partial(
        pl.kernel, _sc_l2_body, mesh=mesh,
        compiler_params=pltpu.CompilerParams(needs_layout_passes=False, use_tc_tiling_on_sc=False),
        out_type=[jax.ShapeDtypeStruct((NC, 2, NP_PAD), _f32),
                  jax.ShapeDtypeStruct((NC, 2, NP_PAD), _f32),
                  jax.ShapeDtypeStruct((NC, 2, NP_PAD), _f32)],
        scratch_types=[
            pltpu.VMEM((N,), _f32),
            pltpu.VMEM((N,), _f32),
            pltpu.VMEM((N, 2), _f32),
            pltpu.VMEM((NCH, CHUNK), _i32),
            pltpu.VMEM((NCH, CHUNK), _i32),
            pltpu.VMEM((CHUNK,), _f32),
            pltpu.VMEM((CHUNK,), _f32),
            pltpu.VMEM((CHUNK,), _f32),
            pltpu.VMEM((640,), _f32),
            pltpu.VMEM((8,), _f32),
            pltpu.VMEM_SHARED((NP_PAD,), _f32),
            pltpu.VMEM_SHARED((NP_PAD,), _f32),
            pltpu.VMEM_SHARED((NP_PAD,), _f32),
            pltpu.VMEM_SHARED((NP_PAD,), _f32),
            pltpu.VMEM_SHARED((NP_PAD,), _f32),
            pltpu.VMEM_SHARED((NP_PAD,), _f32),
        ])()
    return f(h2, prm, src, dst)


def _tca_body(x_ref, W1_ref, as_ref, ad_ref,
              h0_ref, h1_ref, es0_ref, es1_ref, ed0_ref, ed1_ref):
    x = x_ref[...]
    outs = ((h0_ref, es0_ref, ed0_ref), (h1_ref, es1_ref, ed1_ref))
    for m in range(2):
        h = jnp.dot(x, W1_ref[m], preferred_element_type=_f32)
        h_ref, es_ref, ed_ref = outs[m]
        h_ref[...] = h
        es_ref[...] = jnp.dot(h, as_ref[m], preferred_element_type=_f32)
        ed_ref[...] = jnp.dot(h, ad_ref[m], preferred_element_type=_f32)


def _tca(x, W1, a_src1, a_dst1):
    blk = 1000
    grid = N // blk
    outs = [jax.ShapeDtypeStruct((N, HID), _f32)] * 2 + \
           [jax.ShapeDtypeStruct((N, 1), _f32)] * 4
    return pl.pallas_call(
        _tca_body,
        grid=(grid,),
        in_specs=[
            pl.BlockSpec((blk, D), lambda i: (i, 0)),
            pl.BlockSpec((2, D, HID), lambda i: (0, 0, 0)),
            pl.BlockSpec((2, HID, 1), lambda i: (0, 0, 0)),
            pl.BlockSpec((2, HID, 1), lambda i: (0, 0, 0)),
        ],
        out_specs=[pl.BlockSpec((blk, HID), lambda i: (i, 0))] * 2
                  + [pl.BlockSpec((blk, 1), lambda i: (i, 0))] * 4,
        out_shape=outs,
    )(x, W1, a_src1.reshape(2, HID, 1), a_dst1.reshape(2, HID, 1))


def _tcb_body(num_ref, den_ref, b1_ref, g_ref, be_ref, W2_ref, h2_ref):
    num = (num_ref[0, 0] + num_ref[1, 0])[:N]
    den = (den_ref[0, 0, 0] + den_ref[1, 0, 0])[:N]
    dcol = den.reshape(N, 1)
    hid = num / (dcol + 1e-16) + b1_ref[0]
    mean = jnp.mean(hid, axis=0, keepdims=True)
    var = jnp.mean((hid - mean) ** 2, axis=0, keepdims=True)
    hid = (hid - mean) / jnp.sqrt(var + 1e-5) * g_ref[0] + be_ref[0]
    hid = _leaky(hid, 0.01)
    h2_ref[0] = jnp.dot(hid, W2_ref[0], preferred_element_type=_f32)


def _tcb(num1, den1, b1, gamma, beta, W2):
    return pl.pallas_call(
        _tcb_body,
        grid=(2,),
        in_specs=[
            pl.BlockSpec((NC, 1, NP_PAD, HID), lambda m: (0, m, 0, 0)),
            pl.BlockSpec((NC, 1, 1, NP_PAD), lambda m: (0, m, 0, 0)),
            pl.BlockSpec((1, 1, HID), lambda m: (m, 0, 0)),
            pl.BlockSpec((1, 1, HID), lambda m: (m, 0, 0)),
            pl.BlockSpec((1, 1, HID), lambda m: (m, 0, 0)),
            pl.BlockSpec((1, HID, 2), lambda m: (m, 0, 0)),
        ],
        out_specs=pl.BlockSpec((1, N, 2), lambda m: (m, 0, 0)),
        out_shape=jax.ShapeDtypeStruct((2, N, 2), _f32),
    )(num1, den1.reshape(NC, 2, 1, NP_PAD), b1.reshape(2, 1, HID),
      gamma.reshape(2, 1, HID), beta.reshape(2, 1, HID), W2)


def _tcc_body(n0_ref, n1_ref, den_ref, b2_ref, y_ref, mask_ref,
              np_ref, loss_ref):
    y = y_ref[...]
    maskf = mask_ref[...]
    msum = jnp.sum(maskf)
    np_acc = jnp.zeros((N,), _f32)
    loss_acc = jnp.zeros((), _f32)
    for m in range(2):
        den = (den_ref[0, m] + den_ref[1, m])[:N]
        o0 = (n0_ref[0, m] + n0_ref[1, m])[:N] / (den + 1e-16) + b2_ref[m, 0]
        o1 = (n1_ref[0, m] + n1_ref[1, m])[:N] / (den + 1e-16) + b2_ref[m, 1]
        mx = jnp.maximum(o0, o1)
        l0 = o0 - mx
        l1 = o1 - mx
        lse = jnp.log(jnp.exp(l0) + jnp.exp(l1))
        lsm0 = l0 - lse
        lsm1 = l1 - lse
        np_acc = np_acc + jnp.exp(lsm1)
        picked = jnp.where(y == 1, lsm1, lsm0)
        loss_acc = loss_acc - jnp.sum(picked * maskf) / msum
    np_ref[...] = np_acc * 0.5
    loss_ref[...] = jnp.broadcast_to(loss_acc * 0.5, (1, 1))


def _tcc(n0, n1, den2, b2, y, maskf):
    full = lambda *shape: pl.BlockSpec(shape, lambda: tuple(0 for _ in shape))
    return pl.pallas_call(
        _tcc_body,
        in_specs=[full(NC, 2, NP_PAD), full(NC, 2, NP_PAD),
                  full(NC, 2, NP_PAD), full(2, 2), full(N), full(N)],
        out_specs=[full(N), full(1, 1)],
        out_shape=[jax.ShapeDtypeStruct((N,), _f32),
                   jax.ShapeDtypeStruct((1, 1), _f32)],
    )(n0, n1, den2, b2, y, maskf)


def _tcd_body(x_ref, lp_ref, ew_ref, np_ref,
              norm_ref, cs_ref, pw_ref):
    lp = lp_ref[...]
    dp = _sig(lp)
    approx = (jnp.log(dp + EPS) - jnp.log(1.0 - dp + EPS)
              + jnp.log(0.5 + EPS) - jnp.log(1.0 - 0.5 + EPS))
    ao = _sig(approx / 0.1)
    ew = _sig(ew_ref[...])
    cs = (1.0 - ao) * ew
    A = x_ref[...] * cs
    rowsum = jnp.sum(A, axis=1, keepdims=True)
    v = (2.0 * np_ref[...] - 1.0) / (rowsum + 1e-6)
    t = jnp.sum(A * v, axis=0, keepdims=True)
    diff = jnp.sum(A * t, axis=1, keepdims=True)
    norm_ref[...] = _sig(diff)
    cs_ref[...] = cs
    pw_ref[...] = 1.0 - dp


def _tcd(x, logit_p, embedding_w, np_col):
    full = lambda *shape: pl.BlockSpec(shape, lambda: tuple(0 for _ in shape))
    return pl.pallas_call(
        _tcd_body,
        in_specs=[full(N, D), full(1, D), full(1, D), full(N, 1)],
        out_specs=[full(N, 1), full(1, D), full(1, D)],
        out_shape=[jax.ShapeDtypeStruct((N, 1), _f32),
                   jax.ShapeDtypeStruct((1, D), _f32),
                   jax.ShapeDtypeStruct((1, D), _f32)],
    )(x, logit_p.reshape(1, D), embedding_w.reshape(1, D), np_col)


def _gram_body(colscale_ref, xb_ref, xc_ref, out_ref):
    colscale = colscale_ref[...]
    xb = xb_ref[...] * colscale
    xc = xc_ref[...] * colscale
    ones = jnp.ones((1, D), _f32)
    rowsumc = jax.lax.dot_general(ones, xc, (((1,), (1,)), ((), ())),
                                  preferred_element_type=_f32)
    gram = jax.lax.dot_general(xb, xc, (((1,), (1,)), ((), ())),
                               preferred_element_type=_f32)
    out_ref[...] = gram / (rowsumc + 1e-6)


def _gram_pallas(x, colscale):
    gr = (N + ROW_BLK - 1) // ROW_BLK
    gc = (N + COL_BLK - 1) // COL_BLK
    return pl.pallas_call(
        _gram_body,
        grid=(gr, gc),
        in_specs=[
            pl.BlockSpec((1, D), lambda i, j: (0, 0)),
            pl.BlockSpec((ROW_BLK, D), lambda i, j: (i, 0)),
            pl.BlockSpec((COL_BLK, D), lambda i, j: (j, 0)),
        ],
        out_specs=pl.BlockSpec((ROW_BLK, COL_BLK), lambda i, j: (i, j)),
        out_shape=jax.ShapeDtypeStruct((N, N), _f32),
    )(colscale, x, x)


def kernel(x, W1, a_src1, a_dst1, b1, gamma, beta, W2, a_src2, a_dst2, b2,
           embedding_w, logit_p, edge_index_ppi, edge_index_homolog, y,
           train_mask):
    src = jnp.stack([edge_index_ppi[0].reshape(NW, NCH, CHUNK),
                     edge_index_homolog[0].reshape(NW, NCH, CHUNK)])
    dst = jnp.stack([edge_index_ppi[1].reshape(NW, NCH, CHUNK),
                     edge_index_homolog[1].reshape(NW, NCH, CHUNK)])

    h0, h1, es0, es1, ed0, ed1 = _tca(x, W1, a_src1, a_dst1)
    es1c = jnp.stack([es0.reshape(N), es1.reshape(N)])
    ed1c = jnp.stack([ed0.reshape(N), ed1.reshape(N)])

    num1, den1 = _sc_l1(h0, h1, es1c, ed1c, src, dst)

    h2 = _tcb(num1, den1, b1, gamma, beta, W2)
    prm = jnp.concatenate(
        [jnp.zeros((2, 1), _f32), a_src2, a_dst2,
         jnp.zeros((2, 3), _f32)], axis=1)

    n0, n1, den2 = _sc_l2(h2, prm, src, dst)

    node_p, loss2d = _tcc(n0, n1, den2, b2, y,
                          train_mask.astype(_f32))
    loss = loss2d.reshape(1)

    norm_col, colscale, pw2d = _tcd(x, logit_p, embedding_w,
                                    node_p.reshape(N, 1))
    norm1 = norm_col.reshape(N)
    pw_vimp = pw2d.reshape(D)

    graph = _gram_pallas(x, colscale)

    return (node_p, loss, norm1, graph, pw_vimp)


# speedup vs baseline: 42.1175x; 1.0589x over previous
import functools
import jax
import jax.numpy as jnp
import numpy as np
from jax import lax
from jax.experimental import pallas as pl
from jax.experimental.pallas import tpu as pltpu
from jax.experimental.pallas import tpu_sc as plsc

EPS = float(np.finfo(float).eps)
N = 10000
D = 128
HID = 64
E = 320000

NC = 2
NS = 16
NW = NC * NS
EPT = E // NW
CHUNK = 80
NCH = EPT // CHUNK
NPT = N // NS
NP_PAD = 10240
DPT = NP_PAD // NS

ROW_BLK = 1024
COL_BLK = 1024

_f32 = jnp.float32
_i32 = jnp.int32


def _leaky(x, slope):
    return jnp.where(x >= 0, x, slope * x)


def _sig(z):
    return 1.0 / (1.0 + jnp.exp(-z))


def _sc_l1_body(h0_hbm, h1_hbm, es_hbm, ed_hbm, src_hbm, dst_hbm,
                num_out, den_out,
                es_v, ed_v, src_v, dst_v, w_a, w_b, rows_v, rows_b, znum_v,
                zden_v, idx0_v, num_sh, den_sh,
                sem, sem_b, sna, snb, sda, sdb):
    c = lax.axis_index("c")
    s = lax.axis_index("s")
    wid = c * NS + s

    def zrow(r, carry):
        for q in range(4):
            znum_v[r, pl.ds(q * 16, 16)] = jnp.zeros((16,), _f32)
        return carry
    lax.fori_loop(0, 128, zrow, 0)

    def zd(i, carry):
        zden_v[pl.ds(i * 16, 16)] = jnp.zeros((16,), _f32)
        return carry
    lax.fori_loop(0, 40, zd, 0)
    for i in range(CHUNK // 16):
        idx0_v[pl.ds(i * 16, 16)] = jnp.zeros((16,), _i32) + N

    pltpu.async_copy(znum_v.at[pl.ds(0, CHUNK)], num_sh.at[idx0_v], snb,
                     add=True)
    pltpu.async_copy(zden_v.at[pl.ds(0, CHUNK)], den_sh.at[idx0_v], sdb,
                     add=True)

    def wait_num(semref, rows):
        pltpu.make_async_copy(rows, num_sh.at[idx0_v], semref).wait()

    def wait_den(semref, wbuf):
        pltpu.make_async_copy(wbuf, den_sh.at[idx0_v], semref).wait()

    def wait_gather(semref, rows, h_hbm):
        pltpu.make_async_copy(h_hbm.at[src_v.at[0]], rows, semref).wait()

    rsd = pl.ds(s * DPT, DPT)
    for m, h_hbm in enumerate((h0_hbm, h1_hbm)):
        for j in range(5):
            pltpu.sync_copy(znum_v, num_sh.at[pl.ds(s * DPT + j * 128, 128)])
        pltpu.sync_copy(zden_v, den_sh.at[rsd])
        plsc.subcore_barrier()

        pltpu.sync_copy(es_hbm.at[m], es_v)
        pltpu.sync_copy(ed_hbm.at[m], ed_v)
        pltpu.sync_copy(src_hbm.at[m].at[wid], src_v)
        pltpu.sync_copy(dst_hbm.at[m].at[wid], dst_v)

        def scalar_phase(ci, wbuf):
            for i in range(CHUNK // 16):
                sl = pl.ds(i * 16, 16)
                s16 = src_v[ci, sl]
                d16 = dst_v[ci, sl]
                a = (plsc.load_gather(es_v, [s16])
                     + plsc.load_gather(ed_v, [d16]))
                a = jnp.where(a >= 0, a, 0.2 * a)
                wbuf[sl] = jnp.exp(a)

        def scale_rows(rows, wbuf):
            def scale_g(g, carry2):
                base = g * 16
                w16 = wbuf[pl.ds(base, 16)]
                for j in range(16):
                    ws = lax.gather(
                        w16, jnp.full((16, 1), j, _i32),
                        lax.GatherDimensionNumbers(
                            offset_dims=(), collapsed_slice_dims=(0,),
                            start_index_map=(0,)),
                        (1,), mode=lax.GatherScatterMode.PROMISE_IN_BOUNDS)
                    e = base + j
                    for q in range(HID // 16):
                        sl = pl.ds(q * 16, 16)
                        rows[e, sl] = rows[e, sl] * ws
                return carry2
            lax.fori_loop(0, CHUNK // 16, scale_g, 0)

        def scatter(rows, wbuf, ci, sn, sd):
            pltpu.async_copy(rows, num_sh.at[dst_v.at[ci]], sn, add=True)
            pltpu.async_copy(wbuf, den_sh.at[dst_v.at[ci]], sd, add=True)

        cp0 = pltpu.async_copy(h_hbm.at[src_v.at[0]], rows_v, sem)

        def pair_body(k, carry):
            ca = 2 * k
            cb = 2 * k + 1
            scalar_phase(ca, w_a)
            wait_num(snb, rows_b)
            wait_den(sdb, w_b)
            pltpu.async_copy(h_hbm.at[src_v.at[cb]], rows_b, sem_b)
            wait_gather(sem, rows_v, h_hbm)
            scale_rows(rows_v, w_a)
            scatter(rows_v, w_a, ca, sna, sda)
            scalar_phase(cb, w_b)
            wait_num(sna, rows_v)
            wait_den(sda, w_a)
            pltpu.async_copy(h_hbm.at[src_v.at[cb + 1]], rows_v, sem)
            wait_gather(sem_b, rows_b, h_hbm)
            scale_rows(rows_b, w_b)
            scatter(rows_b, w_b, cb, snb, sdb)
            return carry
        lax.fori_loop(0, (NCH - 1) // 2, pair_body, 0)

        scalar_phase(NCH - 1, w_a)
        wait_gather(sem, rows_v, h_hbm)
        scale_rows(rows_v, w_a)
        scatter(rows_v, w_a, NCH - 1, sna, sda)
        wait_num(sna, rows_v)
        wait_den(sda, w_a)
        wait_num(snb, rows_b)
        wait_den(sdb, w_b)
        if m == 0:
            pltpu.async_copy(znum_v.at[pl.ds(0, CHUNK)], num_sh.at[idx0_v],
                             snb, add=True)
            pltpu.async_copy(zden_v.at[pl.ds(0, CHUNK)], den_sh.at[idx0_v],
                             sdb, add=True)
        plsc.subcore_barrier()
        pltpu.sync_copy(num_sh.at[rsd], num_out.at[c].at[m].at[rsd])
        pltpu.sync_copy(den_sh.at[rsd], den_out.at[c].at[m].at[rsd])
        plsc.subcore_barrier()


def _sc_l1(h0, h1, es, ed, src, dst):
    mesh = plsc.VectorSubcoreMesh(core_axis_name="c", subcore_axis_name="s")
    f = functools.partial(
        pl.kernel, _sc_l1_body, mesh=mesh,
        compiler_params=pltpu.CompilerParams(needs_layout_passes=False, use_tc_tiling_on_sc=False),
        out_type=[jax.ShapeDtypeStruct((NC, 2, NP_PAD, HID), _f32),
                  jax.ShapeDtypeStruct((NC, 2, NP_PAD), _f32)],
        scratch_types=[
            pltpu.VMEM((N,), _f32),
            pltpu.VMEM((N,), _f32),
            pltpu.VMEM((NCH, CHUNK), _i32),
            pltpu.VMEM((NCH, CHUNK), _i32),
            pltpu.VMEM((CHUNK,), _f32),
            pltpu.VMEM((CHUNK,), _f32),
            pltpu.VMEM((CHUNK, HID), _f32),
            pltpu.VMEM((CHUNK, HID), _f32),
            pltpu.VMEM((128, HID), _f32),
            pltpu.VMEM((640,), _f32),
            pltpu.VMEM((CHUNK,), _i32),
            pltpu.VMEM_SHARED((NP_PAD, HID), _f32),
            pltpu.VMEM_SHARED((NP_PAD,), _f32),
            pltpu.SemaphoreType.DMA,
            pltpu.SemaphoreType.DMA,
            pltpu.SemaphoreType.DMA,
            pltpu.SemaphoreType.DMA,
            pltpu.SemaphoreType.DMA,
            pltpu.SemaphoreType.DMA,
        ])()
    return f(h0, h1, es, ed, src, dst)


def _sc_l2_body(h2_hbm, prm_hbm, src_hbm, dst_hbm,
                n0_out, n1_out, den_out,
                es_v, ed_v, h2_v, src_v, dst_v,
                w_a, m0_a, m1_a, w_b, m0_b, m1_b, zden_v, idx0_v, prm_v,
                n0a_sh, n0b_sh, n1a_sh, n1b_sh, dena_sh, denb_sh,
                sa, sb):
    c = lax.axis_index("c")
    s = lax.axis_index("s")
    wid = c * NS + s

    def zd(i, carry):
        zden_v[pl.ds(i * 16, 16)] = jnp.zeros((16,), _f32)
        return carry
    lax.fori_loop(0, 40, zd, 0)
    for i in range(CHUNK // 16):
        idx0_v[pl.ds(i * 16, 16)] = jnp.zeros((16,), _i32) + N

    zsl = zden_v.at[pl.ds(0, CHUNK)]

    def credit3(sm):
        pltpu.async_copy(zsl, dena_sh.at[idx0_v], sm, add=True)
        pltpu.async_copy(zsl, dena_sh.at[idx0_v], sm, add=True)
        pltpu.async_copy(zsl, dena_sh.at[idx0_v], sm, add=True)

    def wait3(sm, wbuf):
        for _ in range(3):
            pltpu.make_async_copy(wbuf, dena_sh.at[idx0_v], sm).wait()

    credit3(sa)
    credit3(sb)

    for sh in (n0a_sh, n0b_sh, n1a_sh, n1b_sh, dena_sh, denb_sh):
        pltpu.sync_copy(zden_v, sh.at[pl.ds(s * DPT, DPT)])
    plsc.subcore_barrier()

    z16 = jnp.zeros((16,), _i32)
    iot = lax.iota(_i32, 16)
    for m, (n0_sh, n1_sh, den_sh) in enumerate(
            ((n0a_sh, n1a_sh, dena_sh), (n0b_sh, n1b_sh, denb_sh))):
        pltpu.sync_copy(h2_hbm.at[m], h2_v)
        pltpu.sync_copy(prm_hbm.at[m], prm_v)
        pltpu.sync_copy(src_hbm.at[m].at[wid], src_v)
        pltpu.sync_copy(dst_hbm.at[m].at[wid], dst_v)
        as0 = plsc.load_gather(prm_v, [z16 + 1])
        as1 = plsc.load_gather(prm_v, [z16 + 2])
        ad0 = plsc.load_gather(prm_v, [z16 + 3])
        ad1 = plsc.load_gather(prm_v, [z16 + 4])

        def node_body(i, carry):
            idx16 = iot + i * 16
            c0 = plsc.load_gather(h2_v, [idx16, z16])
            c1 = plsc.load_gather(h2_v, [idx16, z16 + 1])
            sl = pl.ds(i * 16, 16)
            es_v[sl] = as0 * c0 + as1 * c1
            ed_v[sl] = ad0 * c0 + ad1 * c1
            return carry
        lax.fori_loop(0, N // 16, node_body, 0)

        def compute_chunk(ci, wv, m0v, m1v):
            for i in range(CHUNK // 16):
                sl = pl.ds(i * 16, 16)
                s16 = src_v[ci, sl]
                d16 = dst_v[ci, sl]
                a = (plsc.load_gather(es_v, [s16])
                     + plsc.load_gather(ed_v, [d16]))
                a = jnp.where(a >= 0, a, 0.2 * a)
                w16 = jnp.exp(a)
                wv[sl] = w16
                m0v[sl] = w16 * plsc.load_gather(h2_v, [s16, z16])
                m1v[sl] = w16 * plsc.load_gather(h2_v, [s16, z16 + 1])

        def scatter3(ci, wv, m0v, m1v, sm):
            pltpu.async_copy(wv, den_sh.at[dst_v.at[ci]], sm, add=True)
            pltpu.async_copy(m0v, n0_sh.at[dst_v.at[ci]], sm, add=True)
            pltpu.async_copy(m1v, n1_sh.at[dst_v.at[ci]], sm, add=True)

        def pair_body(k, carry):
            ca = 2 * k
            cb = 2 * k + 1
            wait3(sa, w_a)
            compute_chunk(ca, w_a, m0_a, m1_a)
            scatter3(ca, w_a, m0_a, m1_a, sa)
            wait3(sb, w_b)
            compute_chunk(cb, w_b, m0_b, m1_b)
            scatter3(cb, w_b, m0_b, m1_b, sb)
            return carry
        lax.fori_loop(0, (NCH - 1) // 2, pair_body, 0)

        wait3(sa, w_a)
        compute_chunk(NCH - 1, w_a, m0_a, m1_a)
        scatter3(NCH - 1, w_a, m0_a, m1_a, sa)
        wait3(sa, w_a)
        wait3(sb, w_b)
        if m == 0:
            credit3(sa)
            credit3(sb)

    plsc.subcore_barrier()
    rs = pl.ds(s * DPT, DPT)
    pltpu.sync_copy(n0a_sh.at[rs], n0_out.at[c].at[0].at[rs])
    pltpu.sync_copy(n0b_sh.at[rs], n0_out.at[c].at[1].at[rs])
    pltpu.sync_copy(n1a_sh.at[rs], n1_out.at[c].at[0].at[rs])
    pltpu.sync_copy(n1b_sh.at[rs], n1_out.at[c].at[1].at[rs])
    pltpu.sync_copy(dena_sh.at[rs], den_out.at[c].at[0].at[rs])
    pltpu.sync_copy(denb_sh.at[rs], den_out.at[c].at[1].at[rs])


def _sc_l2(h2, prm, src, dst):
    mesh = plsc.VectorSubcoreMesh(core_axis_name="c", subcore_axis_name="s")
    f = functools.partial(
        pl.kernel, _sc_l2_body, mesh=mesh,
        compiler_params=pltpu.CompilerParams(needs_layout_passes=False, use_tc_tiling_on_sc=False),
        out_type=[jax.ShapeDtypeStruct((NC, 2, NP_PAD), _f32),
                  jax.ShapeDtypeStruct((NC, 2, NP_PAD), _f32),
                  jax.ShapeDtypeStruct((NC, 2, NP_PAD), _f32)],
        scratch_types=[
            pltpu.VMEM((N,), _f32),
            pltpu.VMEM((N,), _f32),
            pltpu.VMEM((N, 2), _f32),
            pltpu.VMEM((NCH, CHUNK), _i32),
            pltpu.VMEM((NCH, CHUNK), _i32),
            pltpu.VMEM((CHUNK,), _f32),
            pltpu.VMEM((CHUNK,), _f32),
            pltpu.VMEM((CHUNK,), _f32),
            pltpu.VMEM((CHUNK,), _f32),
            pltpu.VMEM((CHUNK,), _f32),
            pltpu.VMEM((CHUNK,), _f32),
            pltpu.VMEM((640,), _f32),
            pltpu.VMEM((CHUNK,), _i32),
            pltpu.VMEM((8,), _f32),
            pltpu.VMEM_SHARED((NP_PAD,), _f32),
            pltpu.VMEM_SHARED((NP_PAD,), _f32),
            pltpu.VMEM_SHARED((NP_PAD,), _f32),
            pltpu.VMEM_SHARED((NP_PAD,), _f32),
            pltpu.VMEM_SHARED((NP_PAD,), _f32),
            pltpu.VMEM_SHARED((NP_PAD,), _f32),
            pltpu.SemaphoreType.DMA,
            pltpu.SemaphoreType.DMA,
        ])()
    return f(h2, prm, src, dst)


def _tca_body(x_ref, W1_ref, as_ref, ad_ref,
              h0_ref, h1_ref, es0_ref, es1_ref, ed0_ref, ed1_ref):
    x = x_ref[...]
    outs = ((h0_ref, es0_ref, ed0_ref), (h1_ref, es1_ref, ed1_ref))
    for m in range(2):
        h = jnp.dot(x, W1_ref[m], preferred_element_type=_f32)
        h_ref, es_ref, ed_ref = outs[m]
        h_ref[...] = h
        es_ref[...] = jnp.dot(h, as_ref[m], preferred_element_type=_f32)
        ed_ref[...] = jnp.dot(h, ad_ref[m], preferred_element_type=_f32)


def _tca(x, W1, a_src1, a_dst1):
    blk = 1000
    grid = N // blk
    outs = [jax.ShapeDtypeStruct((N, HID), _f32)] * 2 + \
           [jax.ShapeDtypeStruct((N, 1), _f32)] * 4
    return pl.pallas_call(
        _tca_body,
        grid=(grid,),
        in_specs=[
            pl.BlockSpec((blk, D), lambda i: (i, 0)),
            pl.BlockSpec((2, D, HID), lambda i: (0, 0, 0)),
            pl.BlockSpec((2, HID, 1), lambda i: (0, 0, 0)),
            pl.BlockSpec((2, HID, 1), lambda i: (0, 0, 0)),
        ],
        out_specs=[pl.BlockSpec((blk, HID), lambda i: (i, 0))] * 2
                  + [pl.BlockSpec((blk, 1), lambda i: (i, 0))] * 4,
        out_shape=outs,
    )(x, W1, a_src1.reshape(2, HID, 1), a_dst1.reshape(2, HID, 1))


def _tcb_body(num_ref, den_ref, b1_ref, g_ref, be_ref, W2_ref, h2_ref):
    num = (num_ref[0, 0] + num_ref[1, 0])[:N]
    den = (den_ref[0, 0, 0] + den_ref[1, 0, 0])[:N]
    dcol = den.reshape(N, 1)
    hid = num / (dcol + 1e-16) + b1_ref[0]
    mean = jnp.mean(hid, axis=0, keepdims=True)
    var = jnp.mean((hid - mean) ** 2, axis=0, keepdims=True)
    hid = (hid - mean) / jnp.sqrt(var + 1e-5) * g_ref[0] + be_ref[0]
    hid = _leaky(hid, 0.01)
    h2_ref[0] = jnp.dot(hid, W2_ref[0], preferred_element_type=_f32)


def _tcb(num1, den1, b1, gamma, beta, W2):
    return pl.pallas_call(
        _tcb_body,
        grid=(2,),
        in_specs=[
            pl.BlockSpec((NC, 1, NP_PAD, HID), lambda m: (0, m, 0, 0)),
            pl.BlockSpec((NC, 1, 1, NP_PAD), lambda m: (0, m, 0, 0)),
            pl.BlockSpec((1, 1, HID), lambda m: (m, 0, 0)),
            pl.BlockSpec((1, 1, HID), lambda m: (m, 0, 0)),
            pl.BlockSpec((1, 1, HID), lambda m: (m, 0, 0)),
            pl.BlockSpec((1, HID, 2), lambda m: (m, 0, 0)),
        ],
        out_specs=pl.BlockSpec((1, N, 2), lambda m: (m, 0, 0)),
        out_shape=jax.ShapeDtypeStruct((2, N, 2), _f32),
    )(num1, den1.reshape(NC, 2, 1, NP_PAD), b1.reshape(2, 1, HID),
      gamma.reshape(2, 1, HID), beta.reshape(2, 1, HID), W2)


def _tcc_body(n0_ref, n1_ref, den_ref, b2_ref, y_ref, mask_ref,
              np_ref, loss_ref):
    y = y_ref[...]
    maskf = mask_ref[...]
    msum = jnp.sum(maskf)
    np_acc = jnp.zeros((N,), _f32)
    loss_acc = jnp.zeros((), _f32)
    for m in range(2):
        den = (den_ref[0, m] + den_ref[1, m])[:N]
        o0 = (n0_ref[0, m] + n0_ref[1, m])[:N] / (den + 1e-16) + b2_ref[m, 0]
        o1 = (n1_ref[0, m] + n1_ref[1, m])[:N] / (den + 1e-16) + b2_ref[m, 1]
        mx = jnp.maximum(o0, o1)
        l0 = o0 - mx
        l1 = o1 - mx
        lse = jnp.log(jnp.exp(l0) + jnp.exp(l1))
        lsm0 = l0 - lse
        lsm1 = l1 - lse
        np_acc = np_acc + jnp.exp(lsm1)
        picked = jnp.where(y == 1, lsm1, lsm0)
        loss_acc = loss_acc - jnp.sum(picked * maskf) / msum
    np_ref[...] = np_acc * 0.5
    loss_ref[...] = jnp.broadcast_to(loss_acc * 0.5, (1, 1))


def _tcc(n0, n1, den2, b2, y, maskf):
    full = lambda *shape: pl.BlockSpec(shape, lambda: tuple(0 for _ in shape))
    return pl.pallas_call(
        _tcc_body,
        in_specs=[full(NC, 2, NP_PAD), full(NC, 2, NP_PAD),
                  full(NC, 2, NP_PAD), full(2, 2), full(N), full(N)],
        out_specs=[full(N), full(1, 1)],
        out_shape=[jax.ShapeDtypeStruct((N,), _f32),
                   jax.ShapeDtypeStruct((1, 1), _f32)],
    )(n0, n1, den2, b2, y, maskf)


def _tcd_body(x_ref, lp_ref, ew_ref, np_ref,
              norm_ref, cs_ref, pw_ref):
    lp = lp_ref[...]
    dp = _sig(lp)
    approx = (jnp.log(dp + EPS) - jnp.log(1.0 - dp + EPS)
              + jnp.log(0.5 + EPS) - jnp.log(1.0 - 0.5 + EPS))
    ao = _sig(approx / 0.1)
    ew = _sig(ew_ref[...])
    cs = (1.0 - ao) * ew
    A = x_ref[...] * cs
    rowsum = jnp.sum(A, axis=1, keepdims=True)
    v = (2.0 * np_ref[...] - 1.0) / (rowsum + 1e-6)
    t = jnp.sum(A * v, axis=0, keepdims=True)
    diff = jnp.sum(A * t, axis=1, keepdims=True)
    norm_ref[...] = _sig(diff)
    cs_ref[...] = cs
    pw_ref[...] = 1.0 - dp


def _tcd(x, logit_p, embedding_w, np_col):
    full = lambda *shape: pl.BlockSpec(shape, lambda: tuple(0 for _ in shape))
    return pl.pallas_call(
        _tcd_body,
        in_specs=[full(N, D), full(1, D), full(1, D), full(N, 1)],
        out_specs=[full(N, 1), full(1, D), full(1, D)],
        out_shape=[jax.ShapeDtypeStruct((N, 1), _f32),
                   jax.ShapeDtypeStruct((1, D), _f32),
                   jax.ShapeDtypeStruct((1, D), _f32)],
    )(x, logit_p.reshape(1, D), embedding_w.reshape(1, D), np_col)


def _gram_body(colscale_ref, xb_ref, xc_ref, out_ref):
    colscale = colscale_ref[...]
    xb = xb_ref[...] * colscale
    xc = xc_ref[...] * colscale
    ones = jnp.ones((1, D), _f32)
    rowsumc = jax.lax.dot_general(ones, xc, (((1,), (1,)), ((), ())),
                                  preferred_element_type=_f32)
    gram = jax.lax.dot_general(xb, xc, (((1,), (1,)), ((), ())),
                               preferred_element_type=_f32)
    out_ref[...] = gram / (rowsumc + 1e-6)


def _gram_pallas(x, colscale):
    gr = (N + ROW_BLK - 1) // ROW_BLK
    gc = (N + COL_BLK - 1) // COL_BLK
    return pl.pallas_call(
        _gram_body,
        grid=(gr, gc),
        in_specs=[
            pl.BlockSpec((1, D), lambda i, j: (0, 0)),
            pl.BlockSpec((ROW_BLK, D), lambda i, j: (i, 0)),
            pl.BlockSpec((COL_BLK, D), lambda i, j: (j, 0)),
        ],
        out_specs=pl.BlockSpec((ROW_BLK, COL_BLK), lambda i, j: (i, j)),
        out_shape=jax.ShapeDtypeStruct((N, N), _f32),
    )(colscale, x, x)


def kernel(x, W1, a_src1, a_dst1, b1, gamma, beta, W2, a_src2, a_dst2, b2,
           embedding_w, logit_p, edge_index_ppi, edge_index_homolog, y,
           train_mask):
    src = jnp.stack([edge_index_ppi[0].reshape(NW, NCH, CHUNK),
                     edge_index_homolog[0].reshape(NW, NCH, CHUNK)])
    dst = jnp.stack([edge_index_ppi[1].reshape(NW, NCH, CHUNK),
                     edge_index_homolog[1].reshape(NW, NCH, CHUNK)])

    h0, h1, es0, es1, ed0, ed1 = _tca(x, W1, a_src1, a_dst1)
    es1c = jnp.stack([es0.reshape(N), es1.reshape(N)])
    ed1c = jnp.stack([ed0.reshape(N), ed1.reshape(N)])

    num1, den1 = _sc_l1(h0, h1, es1c, ed1c, src, dst)

    h2 = _tcb(num1, den1, b1, gamma, beta, W2)
    prm = jnp.concatenate(
        [jnp.zeros((2, 1), _f32), a_src2, a_dst2,
         jnp.zeros((2, 3), _f32)], axis=1)

    n0, n1, den2 = _sc_l2(h2, prm, src, dst)

    node_p, loss2d = _tcc(n0, n1, den2, b2, y,
                          train_mask.astype(_f32))
    loss = loss2d.reshape(1)

    norm_col, colscale, pw2d = _tcd(x, logit_p, embedding_w,
                                    node_p.reshape(N, 1))
    norm1 = norm_col.reshape(N)
    pw_vimp = pw2d.reshape(D)

    graph = _gram_pallas(x, colscale)

    return (node_p, loss, norm1, graph, pw_vimp)
